# baseline (speedup 1.0000x reference)
import functools
import math

import jax
import jax.numpy as jnp
from jax import lax
from jax.experimental import pallas as pl
from jax.experimental.pallas import tpu as pltpu


def _node_dense_body(srcm, dstm, raw, aux, w1, w2, w3, w4, bih, whh, bhh,
                     mtw, mtb, wq, bq, wk, bk, wv, bv, ws, bs,
                     qn_o, kn_o, vn_o, sk_o):
    d0 = aux[:, 0:1]
    d1 = aux[:, 1:2]
    rt = aux[:, 2:3]
    s = srcm[...]
    dm = dstm[...]
    m1 = s * d0 + dm * d1
    m2 = s * d1 + dm * d0
    te = jnp.cos(rt * mtw[...] + mtb[...])
    f32 = jnp.float32
    gi = (jnp.dot(m1, w1[...], preferred_element_type=f32)
          + jnp.dot(m2, w2[...], preferred_element_type=f32)
          + jnp.dot(raw[...], w3[...], preferred_element_type=f32)
          + jnp.dot(te, w4[...], preferred_element_type=f32) + bih[...])
    gh = jnp.dot(s, whh[...], preferred_element_type=f32) + bhh[...]
    M = 128
    r = jax.nn.sigmoid(gi[:, :M] + gh[:, :M])
    z = jax.nn.sigmoid(gi[:, M:2 * M] + gh[:, M:2 * M])
    n = jnp.tanh(gi[:, 2 * M:] + r * gh[:, 2 * M:])
    x = (1.0 - z) * n + z * s
    qn_o[...] = jnp.dot(x, wq[...], preferred_element_type=f32) + bq[...]
    kn_o[...] = jnp.dot(x, wk[...], preferred_element_type=f32) + bk[...]
    vn_o[...] = jnp.dot(x, wv[...], preferred_element_type=f32) + bv[...]
    sk_o[...] = jnp.dot(x, ws[...], preferred_element_type=f32) + bs[...]


def _node_dense(srcm, dstm, raw, aux, w1, w2, w3, w4, bih, whh, bhh,
                mtw, mtb, wq, bq, wk, bk, wv, bv, ws, bs, NB=512):
    NP = srcm.shape[0]
    grid = (NP // NB,)
    row_spec = lambda c: pl.BlockSpec((NB, c), lambda i: (i, 0))
    full = lambda a: pl.BlockSpec(a.shape, lambda i: (0,) * a.ndim)
    out_shape = [jax.ShapeDtypeStruct((NP, 128), jnp.float32)] * 4
    return pl.pallas_call(
        _node_dense_body,
        grid=grid,
        in_specs=[row_spec(128), row_spec(128), row_spec(100), row_spec(128)]
        + [full(a) for a in (w1, w2, w3, w4, bih, whh, bhh, mtw, mtb,
                             wq, bq, wk, bk, wv, bv, ws, bs)],
        out_specs=[row_spec(128)] * 4,
        out_shape=out_shape,
    )(srcm, dstm, raw, aux, w1, w2, w3, w4, bih, whh, bhh, mtw, mtb,
      wq, bq, wk, bk, wv, bv, ws, bs)


def _edge_dense_body(relt, msg, gtw, gtb, wem, wet, e_o):
    te = jnp.cos(relt[...] * gtw[...] + gtb[...])
    f32 = jnp.float32
    e_o[...] = (jnp.dot(msg[...], wem[...], preferred_element_type=f32)
                + jnp.dot(te, wet[...], preferred_element_type=f32))


def _edge_dense(relt, msg, gtw, gtb, wem, wet, EB=2560):
    E = relt.shape[0]
    grid = (E // EB,)
    full = lambda a: pl.BlockSpec(a.shape, lambda i: (0,) * a.ndim)
    return pl.pallas_call(
        _edge_dense_body,
        grid=grid,
        in_specs=[pl.BlockSpec((EB, 1), lambda i: (i, 0)),
                  pl.BlockSpec((EB, 100), lambda i: (i, 0)),
                  full(gtw), full(gtb), full(wem), full(wet)],
        out_specs=pl.BlockSpec((EB, 128), lambda i: (i, 0)),
        out_shape=jax.ShapeDtypeStruct((E, 128), jnp.float32),
    )(relt, msg, gtw, gtb, wem, wet)


def kernel(mem, mem_msg, direction, msg, mt_w, mt_b, gru_w_ih, gru_w_hh,
           gru_b_ih, gru_b_hh, gt_w, gt_b, wq, bq, wk, bk, wv, bv, we, ws,
           bs, mem_ints, n_id, edge_index, t):
    Nn = n_id.shape[0]
    E = t.shape[0]
    NB = 512
    NP = ((Nn + NB - 1) // NB) * NB

    ints = mem_ints[n_id]
    lu = ints[:, 0]
    rt = ints[:, 1]
    did = ints[:, 2].astype(jnp.int32)
    srcm = mem[n_id]
    dstm = mem[did]
    raw = mem_msg[n_id]
    d = direction[n_id]

    pad = lambda a: jnp.pad(a, ((0, NP - Nn),) + ((0, 0),) * (a.ndim - 1))
    aux = jnp.zeros((NP, 128), jnp.float32)
    aux = aux.at[:Nn, 0].set(d[:, 0]).at[:Nn, 1].set(d[:, 1]).at[:Nn, 2].set(rt)

    wih_t = gru_w_ih.T
    w1 = wih_t[0:128]
    w2 = wih_t[128:256]
    w3 = wih_t[256:356]
    w4 = wih_t[356:456]
    qn, kn, vn, sk = _node_dense(
        pad(srcm), pad(dstm), pad(raw), aux,
        w1, w2, w3, w4, gru_b_ih[None, :], gru_w_hh.T, gru_b_hh[None, :],
        mt_w.T, mt_b[None, :], wq.T, bq[None, :], wk.T, bk[None, :],
        wv.T, bv[None, :], ws.T, bs[None, :])
    qn = qn[:Nn]
    kn = kn[:Nn]
    vn = vn[:Nn]
    sk = sk[:Nn]

    src = edge_index[0]
    dst = edge_index[1]
    relt = lu[src] - t.astype(jnp.float32)
    e = _edge_dense(relt[:, None], msg, gt_w.T, gt_b[None, :], we.T[0:100],
                    we.T[100:200])

    q_e = qn[dst]
    ke = kn[src] + e
    logits = (q_e * ke).reshape(E, 2, 64).sum(-1) * (1.0 / 8.0)
    mx = jnp.max(logits, axis=0)
    ex = jnp.exp(logits - mx)
    s = jnp.zeros((Nn, 2), jnp.float32).at[dst].add(ex)
    ve = (vn[src] + e).reshape(E, 2, 64)
    num = jnp.zeros((Nn, 2, 64), jnp.float32).at[dst].add(ex[:, :, None] * ve)
    out = (num / (s[:, :, None] + 1e-16)).reshape(Nn, 128) + sk
    return out


# speedup vs baseline: 6.3555x; 6.3555x over previous
import functools

import jax
import jax.numpy as jnp
from jax import lax
from jax.experimental import pallas as pl
from jax.experimental.pallas import tpu as pltpu
from jax.experimental.pallas import tpu_sc as plsc

NC = 2
NS = 16
NW = NC * NS
NP = 10240
E_TOT = 320000
EW = E_TOT // NW
CH = 80
NCHUNK = EW // CH
LUT = 10008


def _iota16():
    return lax.iota(jnp.int32, 16)


def _full16(c):
    return jnp.full((16,), c, jnp.int32)


def _sc_gather_body(nid_ref, small_ref, mem_ref, msgt_ref, nidt_ref,
                    luc_ref, didc_ref, srce_ref, tf_ref,
                    small_o, srcm_o, dstm_o, raw_o, relt_o,
                    sidx_v, didx_v, small_v, rows_v, raw_v,
                    nid_v, luc_v, didc_v, eidx_v, tf_v, relt_v, sem):
    wid = lax.axis_index("s") * NC + lax.axis_index("c")
    pltpu.sync_copy(didc_ref, didc_v)

    for b in range(4):
        base = wid * 320 + b * 80
        pltpu.sync_copy(nid_ref.at[pl.ds(base, 80)], sidx_v)
        pltpu.async_copy(small_ref.at[sidx_v], small_v, sem).wait()
        pltpu.sync_copy(small_v, small_o.at[pl.ds(base, 80)])
        for g in range(5):
            nid16 = sidx_v[pl.ds(g * 16, 16)]
            did = plsc.load_gather(didc_v, [nid16])
            didx_v[pl.ds(g * 16, 16)] = did.astype(jnp.int32)
        pltpu.async_copy(mem_ref.at[sidx_v], rows_v, sem).wait()
        pltpu.sync_copy(rows_v, srcm_o.at[pl.ds(base, 80)])
        pltpu.async_copy(mem_ref.at[didx_v], rows_v, sem).wait()
        pltpu.sync_copy(rows_v, dstm_o.at[pl.ds(base, 80)])
        pltpu.async_copy(msgt_ref.at[sidx_v], raw_v, sem).wait()
        pltpu.sync_copy(raw_v, raw_o.at[pl.ds(base, 80)])

    pltpu.sync_copy(nidt_ref, nid_v)
    pltpu.sync_copy(luc_ref, luc_v)
    ECH = 2000

    def echunk(ch, _):
        ebase = wid * EW + ch * ECH
        pltpu.sync_copy(srce_ref.at[pl.ds(ebase, ECH)], eidx_v)
        pltpu.sync_copy(tf_ref.at[pl.ds(ebase, ECH)], tf_v)

        def grp(g, _):
            idx16 = eidx_v[pl.ds(g * 16, 16)]
            j16 = plsc.load_gather(nid_v, [idx16])
            lu16 = plsc.load_gather(luc_v, [j16])
            relt_v[pl.ds(g * 16, 16)] = lu16 - tf_v[pl.ds(g * 16, 16)]
            return 0

        lax.fori_loop(0, ECH // 16, grp, 0)
        pltpu.sync_copy(relt_v, relt_o.at[pl.ds(ebase, ECH)])
        return 0

    lax.fori_loop(0, EW // ECH, echunk, 0)


def _sc_gather(nid_pad, small, mem, msgt, nidt, luc, didc, src_e, tf):
    f32 = jnp.float32
    mesh = plsc.VectorSubcoreMesh(core_axis_name="c", subcore_axis_name="s")
    return pl.kernel(
        _sc_gather_body,
        out_type=[
            jax.ShapeDtypeStruct((NP, 128), f32),
            jax.ShapeDtypeStruct((NP, 128), f32),
            jax.ShapeDtypeStruct((NP, 128), f32),
            jax.ShapeDtypeStruct((NP, 128), f32),
            jax.ShapeDtypeStruct((E_TOT,), f32),
        ],
        mesh=mesh,
        compiler_params=pltpu.CompilerParams(needs_layout_passes=False),
        scratch_types=[
            pltpu.VMEM((80,), jnp.int32),
            pltpu.VMEM((80,), jnp.int32),
            pltpu.VMEM((80, 128), f32),
            pltpu.VMEM((80, 128), f32),
            pltpu.VMEM((80, 128), f32),
            pltpu.VMEM((10000,), jnp.int32),
            pltpu.VMEM((LUT,), f32),
            pltpu.VMEM((LUT,), f32),
            pltpu.VMEM((2000,), jnp.int32),
            pltpu.VMEM((2000,), f32),
            pltpu.VMEM((2000,), f32),
            pltpu.SemaphoreType.DMA,
        ],
    )(nid_pad, small, mem, msgt, nidt, luc, didc, src_e, tf)


def _sc_alpha_body(dst_ref, src_ref, qn_ref, kn_ref, e_ref,
                   a0_o, a1_o,
                   didx_v, sidx_v, q_v, k_v, e_v, a0_v, a1_v, sem, sem2):
    wid = lax.axis_index("s") * NC + lax.axis_index("c")

    def chunk(ch, _):
        ebase = wid * EW + ch * CH
        pltpu.sync_copy(dst_ref.at[pl.ds(ebase, CH)], didx_v)
        pltpu.sync_copy(src_ref.at[pl.ds(ebase, CH)], sidx_v)
        cq = pltpu.async_copy(qn_ref.at[didx_v], q_v, sem)
        ck = pltpu.async_copy(kn_ref.at[sidx_v], k_v, sem2)
        pltpu.sync_copy(e_ref.at[pl.ds(ebase, CH)], e_v)
        cq.wait()
        ck.wait()
        for g in range(CH // 16):
            rows16 = _iota16() + g * 16

            def dot(c4, acc, off):
                for u in range(4):
                    c16 = _full16(off + c4 * 4 + u)
                    q16 = plsc.load_gather(q_v, [rows16, c16])
                    k16 = plsc.load_gather(k_v, [rows16, c16])
                    e16 = plsc.load_gather(e_v, [rows16, c16])
                    acc = acc + q16 * (k16 + e16)
                return acc

            acc0 = lax.fori_loop(0, 16, functools.partial(
                lambda c4, a, o: dot(c4, a, o), o=0),
                jnp.zeros((16,), jnp.float32))
            acc1 = lax.fori_loop(0, 16, functools.partial(
                lambda c4, a, o: dot(c4, a, o), o=64),
                jnp.zeros((16,), jnp.float32))
            a0_v[pl.ds(g * 16, 16)] = acc0 * 0.125
            a1_v[pl.ds(g * 16, 16)] = acc1 * 0.125
        pltpu.sync_copy(a0_v, a0_o.at[pl.ds(ebase, CH)])
        pltpu.sync_copy(a1_v, a1_o.at[pl.ds(ebase, CH)])
        return 0

    lax.fori_loop(0, NCHUNK, chunk, 0)


def _sc_alpha(dst, src, qn, kn, e):
    f32 = jnp.float32
    mesh = plsc.VectorSubcoreMesh(core_axis_name="c", subcore_axis_name="s")
    return pl.kernel(
        _sc_alpha_body,
        out_type=[jax.ShapeDtypeStruct((E_TOT,), f32),
                  jax.ShapeDtypeStruct((E_TOT,), f32)],
        mesh=mesh,
        compiler_params=pltpu.CompilerParams(needs_layout_passes=False),
        scratch_types=[
            pltpu.VMEM((CH,), jnp.int32),
            pltpu.VMEM((CH,), jnp.int32),
            pltpu.VMEM((CH, 128), f32),
            pltpu.VMEM((CH, 128), f32),
            pltpu.VMEM((CH, 128), f32),
            pltpu.VMEM((CH,), f32),
            pltpu.VMEM((CH,), f32),
            pltpu.SemaphoreType.DMA,
            pltpu.SemaphoreType.DMA,
        ],
    )(dst, src, qn, kn, e)


def _sc_scatter_body(dst_ref, src_ref, vn_ref, e_ref, a0_ref, a1_ref,
                     mx_ref, num_o, den_o,
                     didx_v, sidx_v, v_v, e_v, a0_v, a1_v, mx_v,
                     row_v, acc_sh, sem):
    c = lax.axis_index("c")
    s = lax.axis_index("s")
    wid = s * NC + c
    z16 = jnp.zeros((16,), jnp.float32)

    def zero_acc():
        def zrow(r, _):
            for cc in range(8):
                row_v[r, pl.ds(cc * 16, 16)] = z16
            return 0

        lax.fori_loop(0, CH, zrow, 0)
        for j in range(8):
            pltpu.sync_copy(row_v, acc_sh.at[pl.ds(s * 640 + j * 80, 80)])

    zero_acc()
    pltpu.sync_copy(mx_ref, mx_v)
    plsc.subcore_barrier()

    mv = mx_v[pl.ds(0, 16)]
    mx0 = mv[0]
    mx1 = mv[1]

    def chunk1(ch, _):
        ebase = wid * EW + ch * CH
        pltpu.sync_copy(dst_ref.at[pl.ds(ebase, CH)], didx_v)
        pltpu.sync_copy(src_ref.at[pl.ds(ebase, CH)], sidx_v)
        cv = pltpu.async_copy(vn_ref.at[sidx_v], v_v, sem)
        pltpu.sync_copy(e_ref.at[pl.ds(ebase, CH)], e_v)
        pltpu.sync_copy(a0_ref.at[pl.ds(ebase, CH)], a0_v)
        pltpu.sync_copy(a1_ref.at[pl.ds(ebase, CH)], a1_v)
        cv.wait()
        for g in range(CH // 16):
            rows16 = _iota16() + g * 16
            w0 = jnp.exp(a0_v[pl.ds(g * 16, 16)] - mx0)
            w1 = jnp.exp(a1_v[pl.ds(g * 16, 16)] - mx1)

            def col(c4, w, off):
                for u in range(4):
                    c16 = _full16(off + c4 * 4 + u)
                    v16 = plsc.load_gather(v_v, [rows16, c16])
                    e16 = plsc.load_gather(e_v, [rows16, c16])
                    plsc.store_scatter(row_v, [rows16, c16], (v16 + e16) * w)
                return 0

            lax.fori_loop(0, 16, functools.partial(
                lambda c4, _, w, o: col(c4, w, o), w=w0, o=0), 0)
            lax.fori_loop(0, 16, functools.partial(
                lambda c4, _, w, o: col(c4, w, o), w=w1, o=64), 0)
        pltpu.sync_copy(row_v, acc_sh.at[didx_v], add=True)
        return 0

    lax.fori_loop(0, NCHUNK, chunk1, 0)
    plsc.subcore_barrier()
    for j in range(8):
        r0 = s * 640 + j * 80
        pltpu.sync_copy(acc_sh.at[pl.ds(r0, 80)], v_v)
        pltpu.sync_copy(v_v, num_o.at[pl.ds(c * NP + r0, 80)])
    plsc.subcore_barrier()

    zero_acc()
    plsc.subcore_barrier()

    def chunk2(ch, _):
        ebase = wid * EW + ch * CH
        pltpu.sync_copy(dst_ref.at[pl.ds(ebase, CH)], didx_v)
        pltpu.sync_copy(a0_ref.at[pl.ds(ebase, CH)], a0_v)
        pltpu.sync_copy(a1_ref.at[pl.ds(ebase, CH)], a1_v)
        for g in range(CH // 16):
            rows16 = _iota16() + g * 16
            w0 = jnp.exp(a0_v[pl.ds(g * 16, 16)] - mx0)
            w1 = jnp.exp(a1_v[pl.ds(g * 16, 16)] - mx1)
            plsc.store_scatter(row_v, [rows16, _full16(0)], w0)
            plsc.store_scatter(row_v, [rows16, _full16(1)], w1)
        pltpu.sync_copy(row_v, acc_sh.at[didx_v], add=True)
        return 0

    lax.fori_loop(0, NCHUNK, chunk2, 0)
    plsc.subcore_barrier()
    for j in range(8):
        r0 = s * 640 + j * 80
        pltpu.sync_copy(acc_sh.at[pl.ds(r0, 80)], v_v)
        pltpu.sync_copy(v_v, den_o.at[pl.ds(c * NP + r0, 80)])


def _sc_scatter(dst, src, vn, e, a0, a1, mx16):
    f32 = jnp.float32
    mesh = plsc.VectorSubcoreMesh(core_axis_name="c", subcore_axis_name="s")
    return pl.kernel(
        _sc_scatter_body,
        out_type=[jax.ShapeDtypeStruct((2 * NP, 128), f32),
                  jax.ShapeDtypeStruct((2 * NP, 128), f32)],
        mesh=mesh,
        compiler_params=pltpu.CompilerParams(needs_layout_passes=False),
        scratch_types=[
            pltpu.VMEM((CH,), jnp.int32),
            pltpu.VMEM((CH,), jnp.int32),
            pltpu.VMEM((CH, 128), f32),
            pltpu.VMEM((CH, 128), f32),
            pltpu.VMEM((CH,), f32),
            pltpu.VMEM((CH,), f32),
            pltpu.VMEM((16,), f32),
            pltpu.VMEM((CH, 128), f32),
            pltpu.VMEM_SHARED((NP, 128), f32),
            pltpu.SemaphoreType.DMA,
        ],
    )(dst, src, vn, e, a0, a1, mx16)


def _node_dense_body(srcm, dstm, raw, aux, w1, w2, w3, w4, bih, whh, bhh,
                     mtw, mtb, wq, bq, wk, bk, wv, bv, ws, bs,
                     qn_o, kn_o, vn_o, sk_o):
    d0 = aux[:, 0:1]
    d1 = aux[:, 1:2]
    rt = aux[:, 3:4]
    s = srcm[...]
    dm = dstm[...]
    m1 = s * d0 + dm * d1
    m2 = s * d1 + dm * d0
    te = jnp.cos(rt * mtw[...] + mtb[...])
    f32 = jnp.float32
    gi = (jnp.dot(m1, w1[...], preferred_element_type=f32)
          + jnp.dot(m2, w2[...], preferred_element_type=f32)
          + jnp.dot(raw[...], w3[...], preferred_element_type=f32)
          + jnp.dot(te, w4[...], preferred_element_type=f32) + bih[...])
    gh = jnp.dot(s, whh[...], preferred_element_type=f32) + bhh[...]
    M = 128
    r = jax.nn.sigmoid(gi[:, :M] + gh[:, :M])
    z = jax.nn.sigmoid(gi[:, M:2 * M] + gh[:, M:2 * M])
    n = jnp.tanh(gi[:, 2 * M:] + r * gh[:, 2 * M:])
    x = (1.0 - z) * n + z * s
    qn_o[...] = jnp.dot(x, wq[...], preferred_element_type=f32) + bq[...]
    kn_o[...] = jnp.dot(x, wk[...], preferred_element_type=f32) + bk[...]
    vn_o[...] = jnp.dot(x, wv[...], preferred_element_type=f32) + bv[...]
    sk_o[...] = jnp.dot(x, ws[...], preferred_element_type=f32) + bs[...]


def _node_dense(srcm, dstm, raw, aux, w1, w2, w3, w4, bih, whh, bhh,
                mtw, mtb, wq, bq, wk, bk, wv, bv, ws, bs, NB=512):
    grid = (NP // NB,)
    row_spec = lambda c: pl.BlockSpec((NB, c), lambda i: (i, 0))
    full = lambda a: pl.BlockSpec(a.shape, lambda i: (0,) * a.ndim)
    out_shape = [jax.ShapeDtypeStruct((NP, 128), jnp.float32)] * 4
    return pl.pallas_call(
        _node_dense_body,
        grid=grid,
        in_specs=[row_spec(128), row_spec(128), row_spec(128), row_spec(128)]
        + [full(a) for a in (w1, w2, w3, w4, bih, whh, bhh, mtw, mtb,
                             wq, bq, wk, bk, wv, bv, ws, bs)],
        out_specs=[row_spec(128)] * 4,
        out_shape=out_shape,
    )(srcm, dstm, raw, aux, w1, w2, w3, w4, bih, whh, bhh, mtw, mtb,
      wq, bq, wk, bk, wv, bv, ws, bs)


def _edge_dense_body(relt, msg, gtw, gtb, wem, wet, e_o):
    te = jnp.cos(relt[...] * gtw[...] + gtb[...])
    f32 = jnp.float32
    e_o[...] = (jnp.dot(msg[...], wem[...], preferred_element_type=f32)
                + jnp.dot(te, wet[...], preferred_element_type=f32))


def _edge_dense(relt, msg, gtw, gtb, wem, wet, EB=2560):
    grid = (E_TOT // EB,)
    full = lambda a: pl.BlockSpec(a.shape, lambda i: (0,) * a.ndim)
    return pl.pallas_call(
        _edge_dense_body,
        grid=grid,
        in_specs=[pl.BlockSpec((EB, 1), lambda i: (i, 0)),
                  pl.BlockSpec((EB, 100), lambda i: (i, 0)),
                  full(gtw), full(gtb), full(wem), full(wet)],
        out_specs=pl.BlockSpec((EB, 128), lambda i: (i, 0)),
        out_shape=jax.ShapeDtypeStruct((E_TOT, 128), jnp.float32),
    )(relt, msg, gtw, gtb, wem, wet)


def _max_body(a0, a1, mx_o):
    i = lax.broadcasted_iota(jnp.int32, (1, 128), 1)
    m0 = jnp.max(a0[...])
    m1 = jnp.max(a1[...])
    mx_o[...] = jnp.where(i == 0, m0, jnp.where(i == 1, m1, 0.0))


def _max_tc(a0r, a1r):
    full = lambda a: pl.BlockSpec(a.shape, lambda: (0,) * a.ndim)
    return pl.pallas_call(
        _max_body,
        in_specs=[full(a0r), full(a1r)],
        out_specs=pl.BlockSpec((1, 128), lambda: (0, 0)),
        out_shape=jax.ShapeDtypeStruct((1, 128), jnp.float32),
    )(a0r, a1r)


def _final_body(n0, n1, d0, d1, sk, out_o):
    num = n0[...] + n1[...]
    den0 = d0[:, 0:1] + d1[:, 0:1]
    den1 = d0[:, 1:2] + d1[:, 1:2]
    NB = num.shape[0]
    den = jnp.concatenate([jnp.broadcast_to(den0, (NB, 64)),
                           jnp.broadcast_to(den1, (NB, 64))], axis=1)
    out_o[...] = num / (den + 1e-16) + sk[...]


def _final_tc(num, den, sk, NB=512):
    grid = (NP // NB,)
    nblk = NP // NB
    return pl.pallas_call(
        _final_body,
        grid=grid,
        in_specs=[pl.BlockSpec((NB, 128), lambda i: (i, 0)),
                  pl.BlockSpec((NB, 128), lambda i, n=nblk: (i + n, 0)),
                  pl.BlockSpec((NB, 128), lambda i: (i, 0)),
                  pl.BlockSpec((NB, 128), lambda i, n=nblk: (i + n, 0)),
                  pl.BlockSpec((NB, 128), lambda i: (i, 0))],
        out_specs=pl.BlockSpec((NB, 128), lambda i: (i, 0)),
        out_shape=jax.ShapeDtypeStruct((NP, 128), jnp.float32),
    )(num, num, den, den, sk)


def kernel(mem, mem_msg, direction, msg, mt_w, mt_b, gru_w_ih, gru_w_hh,
           gru_b_ih, gru_b_hh, gt_w, gt_b, wq, bq, wk, bk, wv, bv, we, ws,
           bs, mem_ints, n_id, edge_index, t):
    Nn = n_id.shape[0]
    f32 = jnp.float32

    nid_pad = jnp.pad(n_id.astype(jnp.int32), (0, NP - Nn))
    small = jnp.pad(jnp.concatenate([direction, mem_ints], axis=1),
                    ((0, 0), (0, 123)))
    msgt = jnp.pad(mem_msg, ((0, 0), (0, 28)))
    luc = jnp.pad(mem_ints[:, 0], (0, LUT - mem_ints.shape[0]))
    didc = jnp.pad(mem_ints[:, 2], (0, LUT - mem_ints.shape[0]))
    src = edge_index[0].astype(jnp.int32)
    dst = edge_index[1].astype(jnp.int32)
    tf = t.astype(f32)

    small_g, srcm, dstm, raw, relt = _sc_gather(
        nid_pad, small, mem, msgt, n_id.astype(jnp.int32), luc, didc,
        src, tf)

    wih_t = gru_w_ih.T
    w3 = jnp.pad(wih_t[256:356], ((0, 28), (0, 0)))
    qn, kn, vn, sk = _node_dense(
        srcm, dstm, raw, small_g,
        wih_t[0:128], wih_t[128:256], w3, wih_t[356:456],
        gru_b_ih[None, :], gru_w_hh.T, gru_b_hh[None, :],
        mt_w.T, mt_b[None, :], wq.T, bq[None, :], wk.T, bk[None, :],
        wv.T, bv[None, :], ws.T, bs[None, :])

    e = _edge_dense(relt[:, None], msg, gt_w.T, gt_b[None, :],
                    we.T[0:100], we.T[100:200])

    a0, a1 = _sc_alpha(dst, src, qn, kn, e)

    mx = _max_tc(a0.reshape(2500, 128), a1.reshape(2500, 128))
    mx16 = mx[0, 0:16]

    num, den = _sc_scatter(dst, src, vn, e, a0, a1, mx16)

    out = _final_tc(num, den, sk)
    return out[:Nn]


# speedup vs baseline: 6.8220x; 1.0734x over previous
import functools

import jax
import jax.numpy as jnp
from jax import lax
from jax.experimental import pallas as pl
from jax.experimental.pallas import tpu as pltpu
from jax.experimental.pallas import tpu_sc as plsc

NC = 2
NS = 16
NW = NC * NS
NP = 10240
E_TOT = 320000
EW = E_TOT // NW
CH = 80
NCHUNK = EW // CH
LUT = 10008


def _iota16():
    return lax.iota(jnp.int32, 16)


def _full16(c):
    return jnp.full((16,), c, jnp.int32)


def _sc_gather_body(nid_ref, small_ref, mem_ref, msgt_ref, nidt_ref,
                    luc_ref, didc_ref, srce_ref, tf_ref,
                    small_o, srcm_o, dstm_o, raw_o, relt_o,
                    sidx_v, didx_v, small_v, rows_v, raw_v,
                    nid_v, luc_v, didc_v, eidx_v, tf_v, relt_v, sem):
    wid = lax.axis_index("s") * NC + lax.axis_index("c")
    pltpu.sync_copy(didc_ref, didc_v)

    for b in range(4):
        base = wid * 320 + b * 80
        pltpu.sync_copy(nid_ref.at[pl.ds(base, 80)], sidx_v)
        pltpu.async_copy(small_ref.at[sidx_v], small_v, sem).wait()
        pltpu.sync_copy(small_v, small_o.at[pl.ds(base, 80)])
        for g in range(5):
            nid16 = sidx_v[pl.ds(g * 16, 16)]
            did = plsc.load_gather(didc_v, [nid16])
            didx_v[pl.ds(g * 16, 16)] = did.astype(jnp.int32)
        pltpu.async_copy(mem_ref.at[sidx_v], rows_v, sem).wait()
        pltpu.sync_copy(rows_v, srcm_o.at[pl.ds(base, 80)])
        pltpu.async_copy(mem_ref.at[didx_v], rows_v, sem).wait()
        pltpu.sync_copy(rows_v, dstm_o.at[pl.ds(base, 80)])
        pltpu.async_copy(msgt_ref.at[sidx_v], raw_v, sem).wait()
        pltpu.sync_copy(raw_v, raw_o.at[pl.ds(base, 80)])

    pltpu.sync_copy(nidt_ref, nid_v)
    pltpu.sync_copy(luc_ref, luc_v)
    ECH = 2000

    def echunk(ch, _):
        ebase = wid * EW + ch * ECH
        pltpu.sync_copy(srce_ref.at[pl.ds(ebase, ECH)], eidx_v)
        pltpu.sync_copy(tf_ref.at[pl.ds(ebase, ECH)], tf_v)

        def grp(g, _):
            idx16 = eidx_v[pl.ds(g * 16, 16)]
            j16 = plsc.load_gather(nid_v, [idx16])
            lu16 = plsc.load_gather(luc_v, [j16])
            relt_v[pl.ds(g * 16, 16)] = lu16 - tf_v[pl.ds(g * 16, 16)]
            return 0

        lax.fori_loop(0, ECH // 16, grp, 0)
        pltpu.sync_copy(relt_v, relt_o.at[pl.ds(ebase, ECH)])
        return 0

    lax.fori_loop(0, EW // ECH, echunk, 0)


def _sc_gather(nid_pad, small, mem, msgt, nidt, luc, didc, src_e, tf):
    f32 = jnp.float32
    mesh = plsc.VectorSubcoreMesh(core_axis_name="c", subcore_axis_name="s")
    return pl.kernel(
        _sc_gather_body,
        out_type=[
            jax.ShapeDtypeStruct((NP, 128), f32),
            jax.ShapeDtypeStruct((NP, 128), f32),
            jax.ShapeDtypeStruct((NP, 128), f32),
            jax.ShapeDtypeStruct((NP, 128), f32),
            jax.ShapeDtypeStruct((E_TOT,), f32),
        ],
        mesh=mesh,
        compiler_params=pltpu.CompilerParams(needs_layout_passes=False),
        scratch_types=[
            pltpu.VMEM((80,), jnp.int32),
            pltpu.VMEM((80,), jnp.int32),
            pltpu.VMEM((80, 128), f32),
            pltpu.VMEM((80, 128), f32),
            pltpu.VMEM((80, 128), f32),
            pltpu.VMEM((10000,), jnp.int32),
            pltpu.VMEM((LUT,), f32),
            pltpu.VMEM((LUT,), f32),
            pltpu.VMEM((2000,), jnp.int32),
            pltpu.VMEM((2000,), f32),
            pltpu.VMEM((2000,), f32),
            pltpu.SemaphoreType.DMA,
        ],
    )(nid_pad, small, mem, msgt, nidt, luc, didc, src_e, tf)


def _sc_alpha_body(dst_ref, src_ref, qn_ref, kn_ref, e_ref,
                   a0_o, a1_o,
                   didx0, sidx0, q0, k0, e0,
                   didx1, sidx1, q1, k1, e1,
                   a0_v, a1_v, semq0, semk0, seme0, semq1, semk1, seme1):
    wid = lax.axis_index("s") * NC + lax.axis_index("c")
    bufs = ((didx0, sidx0, q0, k0, e0, semq0, semk0, seme0),
            (didx1, sidx1, q1, k1, e1, semq1, semk1, seme1))

    def prefetch(ch, b):
        didx_v, sidx_v, q_v, k_v, e_v, sq, sk_, se = bufs[b]
        ebase = wid * EW + ch * CH
        pltpu.sync_copy(dst_ref.at[pl.ds(ebase, CH)], didx_v)
        pltpu.sync_copy(src_ref.at[pl.ds(ebase, CH)], sidx_v)
        pltpu.async_copy(qn_ref.at[didx_v], q_v, sq)
        pltpu.async_copy(kn_ref.at[sidx_v], k_v, sk_)
        pltpu.async_copy(e_ref.at[pl.ds(ebase, CH)], e_v, se)

    def wait(b):
        didx_v, sidx_v, q_v, k_v, e_v, sq, sk_, se = bufs[b]
        pltpu.make_async_copy(qn_ref.at[didx_v], q_v, sq).wait()
        pltpu.make_async_copy(kn_ref.at[sidx_v], k_v, sk_).wait()
        pltpu.make_async_copy(e_ref.at[pl.ds(0, CH)], e_v, se).wait()

    def compute(ch, b):
        _, _, q_v, k_v, e_v, _, _, _ = bufs[b]
        ebase = wid * EW + ch * CH
        for g in range(CH // 16):
            rows16 = _iota16() + g * 16

            def dot(c4, acc, off):
                for u in range(4):
                    c16 = _full16(off + c4 * 4 + u)
                    q16 = plsc.load_gather(q_v, [rows16, c16])
                    k16 = plsc.load_gather(k_v, [rows16, c16])
                    e16 = plsc.load_gather(e_v, [rows16, c16])
                    acc = acc + q16 * (k16 + e16)
                return acc

            acc0 = lax.fori_loop(0, 16, functools.partial(
                lambda c4, a, o: dot(c4, a, o), o=0),
                jnp.zeros((16,), jnp.float32))
            acc1 = lax.fori_loop(0, 16, functools.partial(
                lambda c4, a, o: dot(c4, a, o), o=64),
                jnp.zeros((16,), jnp.float32))
            a0_v[pl.ds(g * 16, 16)] = acc0 * 0.125
            a1_v[pl.ds(g * 16, 16)] = acc1 * 0.125
        pltpu.sync_copy(a0_v, a0_o.at[pl.ds(ebase, CH)])
        pltpu.sync_copy(a1_v, a1_o.at[pl.ds(ebase, CH)])

    prefetch(0, 0)

    def pair(p, _):
        ch = p * 2
        prefetch2 = ch + 1
        prefetch(prefetch2, 1)
        wait(0)
        compute(ch, 0)
        prefetch(ch + 2, 0)
        wait(1)
        compute(ch + 1, 1)
        return 0

    lax.fori_loop(0, (NCHUNK - 1) // 2, pair, 0)
    wait(0)
    compute(NCHUNK - 1, 0)


def _sc_alpha(dst, src, qn, kn, e):
    f32 = jnp.float32
    mesh = plsc.VectorSubcoreMesh(core_axis_name="c", subcore_axis_name="s")
    return pl.kernel(
        _sc_alpha_body,
        out_type=[jax.ShapeDtypeStruct((E_TOT,), f32),
                  jax.ShapeDtypeStruct((E_TOT,), f32)],
        mesh=mesh,
        compiler_params=pltpu.CompilerParams(needs_layout_passes=False),
        scratch_types=[
            pltpu.VMEM((CH,), jnp.int32),
            pltpu.VMEM((CH,), jnp.int32),
            pltpu.VMEM((CH, 128), f32),
            pltpu.VMEM((CH, 128), f32),
            pltpu.VMEM((CH, 128), f32),
            pltpu.VMEM((CH,), jnp.int32),
            pltpu.VMEM((CH,), jnp.int32),
            pltpu.VMEM((CH, 128), f32),
            pltpu.VMEM((CH, 128), f32),
            pltpu.VMEM((CH, 128), f32),
            pltpu.VMEM((CH,), f32),
            pltpu.VMEM((CH,), f32),
            pltpu.SemaphoreType.DMA,
            pltpu.SemaphoreType.DMA,
            pltpu.SemaphoreType.DMA,
            pltpu.SemaphoreType.DMA,
            pltpu.SemaphoreType.DMA,
            pltpu.SemaphoreType.DMA,
        ],
    )(dst, src, qn, kn, e)


def _sc_scatter_body(dst_ref, src_ref, vn_ref, e_ref, a0_ref, a1_ref,
                     mx_ref, num_o, den_o,
                     didx0, sidx0, v0, e0, a00, a10,
                     didx1, sidx1, v1, e1, a01, a11,
                     mx_v, acc_sh,
                     semv0, seme0, semsc0, semv1, seme1, semsc1):
    c = lax.axis_index("c")
    s = lax.axis_index("s")
    wid = s * NC + c
    z16 = jnp.zeros((16,), jnp.float32)
    bufs = ((didx0, sidx0, v0, e0, a00, a10, e0, semv0, seme0, semsc0),
            (didx1, sidx1, v1, e1, a01, a11, e1, semv1, seme1, semsc1))

    def zero_rows():
        def zrow(r, _):
            for cc in range(8):
                e0[r, pl.ds(cc * 16, 16)] = z16
                e1[r, pl.ds(cc * 16, 16)] = z16
            return 0

        lax.fori_loop(0, CH, zrow, 0)

    def zero_acc():
        for j in range(8):
            pltpu.sync_copy(e0, acc_sh.at[pl.ds(s * 640 + j * 80, 80)])

    zero_rows()
    zero_acc()
    pltpu.sync_copy(mx_ref, mx_v)
    plsc.subcore_barrier()

    mv = mx_v[pl.ds(0, 16)]
    mx0 = mv[0]
    mx1 = mv[1]

    def prefetch1(ch, b):
        didx_v, sidx_v, v_v, e_v, a0_v, a1_v, _, sv, se, _ = bufs[b]
        ebase = wid * EW + ch * CH
        pltpu.sync_copy(dst_ref.at[pl.ds(ebase, CH)], didx_v)
        pltpu.sync_copy(src_ref.at[pl.ds(ebase, CH)], sidx_v)
        pltpu.async_copy(vn_ref.at[sidx_v], v_v, sv)
        pltpu.async_copy(e_ref.at[pl.ds(ebase, CH)], e_v, se)
        pltpu.sync_copy(a0_ref.at[pl.ds(ebase, CH)], a0_v)
        pltpu.sync_copy(a1_ref.at[pl.ds(ebase, CH)], a1_v)

    def wait1(b):
        didx_v, sidx_v, v_v, e_v, _, _, _, sv, se, _ = bufs[b]
        pltpu.make_async_copy(vn_ref.at[sidx_v], v_v, sv).wait()
        pltpu.make_async_copy(e_ref.at[pl.ds(0, CH)], e_v, se).wait()

    def compute1(b):
        _, _, v_v, e_v, a0_v, a1_v, row_v, _, _, _ = bufs[b]
        for g in range(CH // 16):
            rows16 = _iota16() + g * 16
            w0 = jnp.exp(a0_v[pl.ds(g * 16, 16)] - mx0)
            w1 = jnp.exp(a1_v[pl.ds(g * 16, 16)] - mx1)

            def col(c4, w, off):
                for u in range(4):
                    c16 = _full16(off + c4 * 4 + u)
                    v16 = plsc.load_gather(v_v, [rows16, c16])
                    e16 = plsc.load_gather(e_v, [rows16, c16])
                    plsc.store_scatter(row_v, [rows16, c16], (v16 + e16) * w)
                return 0

            lax.fori_loop(0, 16, functools.partial(
                lambda c4, _, w, o: col(c4, w, o), w=w0, o=0), 0)
            lax.fori_loop(0, 16, functools.partial(
                lambda c4, _, w, o: col(c4, w, o), w=w1, o=64), 0)

    def scatter(b):
        didx_v, _, _, _, _, _, row_v, _, _, ssc = bufs[b]
        pltpu.async_copy(row_v, acc_sh.at[didx_v], ssc, add=True)

    def wait_sc(b):
        didx_v, _, _, _, _, _, row_v, _, _, ssc = bufs[b]
        pltpu.make_async_copy(row_v, acc_sh.at[didx_v], ssc).wait()

    prefetch1(0, 0)

    def pair1(p, _):
        for u in range(2):
            ch = p * 2 + u
            wait1(u)
            compute1(u)
            scatter(u)
            if u == 0:
                pl.when(p >= 1)(lambda: wait_sc(1))
            else:
                wait_sc(0)
            prefetch1(ch + 1, 1 - u)
        return 0

    lax.fori_loop(0, (NCHUNK - 1) // 2, pair1, 0)
    wait1(0)
    compute1(0)
    scatter(0)
    wait_sc(1)
    wait_sc(0)
    plsc.subcore_barrier()
    for j in range(8):
        r0 = s * 640 + j * 80
        pltpu.sync_copy(acc_sh.at[pl.ds(r0, 80)], v0)
        pltpu.sync_copy(v0, num_o.at[pl.ds(c * NP + r0, 80)])
    plsc.subcore_barrier()

    zero_rows()
    zero_acc()
    plsc.subcore_barrier()

    def prefetch2(ch, b):
        didx_v, _, _, _, a0_v, a1_v, _, _, _, _ = bufs[b]
        ebase = wid * EW + ch * CH
        pltpu.sync_copy(dst_ref.at[pl.ds(ebase, CH)], didx_v)
        pltpu.sync_copy(a0_ref.at[pl.ds(ebase, CH)], a0_v)
        pltpu.sync_copy(a1_ref.at[pl.ds(ebase, CH)], a1_v)

    def compute2(b):
        _, _, _, _, a0_v, a1_v, row_v, _, _, _ = bufs[b]
        for g in range(CH // 16):
            rows16 = _iota16() + g * 16
            w0 = jnp.exp(a0_v[pl.ds(g * 16, 16)] - mx0)
            w1 = jnp.exp(a1_v[pl.ds(g * 16, 16)] - mx1)
            plsc.store_scatter(row_v, [rows16, _full16(0)], w0)
            plsc.store_scatter(row_v, [rows16, _full16(1)], w1)

    prefetch2(0, 0)

    def pair2(p, _):
        for u in range(2):
            ch = p * 2 + u
            compute2(u)
            scatter(u)
            if u == 0:
                pl.when(p >= 1)(lambda: wait_sc(1))
            else:
                wait_sc(0)
            prefetch2(ch + 1, 1 - u)
        return 0

    lax.fori_loop(0, (NCHUNK - 1) // 2, pair2, 0)
    compute2(0)
    scatter(0)
    wait_sc(1)
    wait_sc(0)
    plsc.subcore_barrier()
    for j in range(8):
        r0 = s * 640 + j * 80
        pltpu.sync_copy(acc_sh.at[pl.ds(r0, 80)], v0)
        pltpu.sync_copy(v0, den_o.at[pl.ds(c * NP + r0, 80)])


def _sc_scatter(dst, src, vn, e, a0, a1, mx16):
    f32 = jnp.float32
    i32 = jnp.int32
    mesh = plsc.VectorSubcoreMesh(core_axis_name="c", subcore_axis_name="s")
    buf = [pltpu.VMEM((CH,), i32), pltpu.VMEM((CH,), i32),
           pltpu.VMEM((CH, 128), f32), pltpu.VMEM((CH, 128), f32),
           pltpu.VMEM((CH,), f32), pltpu.VMEM((CH,), f32)]
    return pl.kernel(
        _sc_scatter_body,
        out_type=[jax.ShapeDtypeStruct((2 * NP, 128), f32),
                  jax.ShapeDtypeStruct((2 * NP, 128), f32)],
        mesh=mesh,
        compiler_params=pltpu.CompilerParams(needs_layout_passes=False),
        scratch_types=buf + buf + [
            pltpu.VMEM((16,), f32),
            pltpu.VMEM_SHARED((NP, 128), f32),
            pltpu.SemaphoreType.DMA,
            pltpu.SemaphoreType.DMA,
            pltpu.SemaphoreType.DMA,
            pltpu.SemaphoreType.DMA,
            pltpu.SemaphoreType.DMA,
            pltpu.SemaphoreType.DMA,
        ],
    )(dst, src, vn, e, a0, a1, mx16)


def _node_dense_body(srcm, dstm, raw, aux, w1, w2, w3, w4, bih, whh, bhh,
                     mtw, mtb, wq, bq, wk, bk, wv, bv, ws, bs,
                     qn_o, kn_o, vn_o, sk_o):
    d0 = aux[:, 0:1]
    d1 = aux[:, 1:2]
    rt = aux[:, 3:4]
    s = srcm[...]
    dm = dstm[...]
    m1 = s * d0 + dm * d1
    m2 = s * d1 + dm * d0
    te = jnp.cos(rt * mtw[...] + mtb[...])
    f32 = jnp.float32
    gi = (jnp.dot(m1, w1[...], preferred_element_type=f32)
          + jnp.dot(m2, w2[...], preferred_element_type=f32)
          + jnp.dot(raw[...], w3[...], preferred_element_type=f32)
          + jnp.dot(te, w4[...], preferred_element_type=f32) + bih[...])
    gh = jnp.dot(s, whh[...], preferred_element_type=f32) + bhh[...]
    M = 128
    r = jax.nn.sigmoid(gi[:, :M] + gh[:, :M])
    z = jax.nn.sigmoid(gi[:, M:2 * M] + gh[:, M:2 * M])
    n = jnp.tanh(gi[:, 2 * M:] + r * gh[:, 2 * M:])
    x = (1.0 - z) * n + z * s
    qn_o[...] = jnp.dot(x, wq[...], preferred_element_type=f32) + bq[...]
    kn_o[...] = jnp.dot(x, wk[...], preferred_element_type=f32) + bk[...]
    vn_o[...] = jnp.dot(x, wv[...], preferred_element_type=f32) + bv[...]
    sk_o[...] = jnp.dot(x, ws[...], preferred_element_type=f32) + bs[...]


def _node_dense(srcm, dstm, raw, aux, w1, w2, w3, w4, bih, whh, bhh,
                mtw, mtb, wq, bq, wk, bk, wv, bv, ws, bs, NB=512):
    grid = (NP // NB,)
    row_spec = lambda c: pl.BlockSpec((NB, c), lambda i: (i, 0))
    full = lambda a: pl.BlockSpec(a.shape, lambda i: (0,) * a.ndim)
    out_shape = [jax.ShapeDtypeStruct((NP, 128), jnp.float32)] * 4
    return pl.pallas_call(
        _node_dense_body,
        grid=grid,
        in_specs=[row_spec(128), row_spec(128), row_spec(128), row_spec(128)]
        + [full(a) for a in (w1, w2, w3, w4, bih, whh, bhh, mtw, mtb,
                             wq, bq, wk, bk, wv, bv, ws, bs)],
        out_specs=[row_spec(128)] * 4,
        out_shape=out_shape,
    )(srcm, dstm, raw, aux, w1, w2, w3, w4, bih, whh, bhh, mtw, mtb,
      wq, bq, wk, bk, wv, bv, ws, bs)


def _edge_dense_body(relt, msg, gtw, gtb, wem, wet, e_o):
    te = jnp.cos(relt[...] * gtw[...] + gtb[...])
    f32 = jnp.float32
    e_o[...] = (jnp.dot(msg[...], wem[...], preferred_element_type=f32)
                + jnp.dot(te, wet[...], preferred_element_type=f32))


def _edge_dense(relt, msg, gtw, gtb, wem, wet, EB=2560):
    grid = (E_TOT // EB,)
    full = lambda a: pl.BlockSpec(a.shape, lambda i: (0,) * a.ndim)
    return pl.pallas_call(
        _edge_dense_body,
        grid=grid,
        in_specs=[pl.BlockSpec((EB, 1), lambda i: (i, 0)),
                  pl.BlockSpec((EB, 100), lambda i: (i, 0)),
                  full(gtw), full(gtb), full(wem), full(wet)],
        out_specs=pl.BlockSpec((EB, 128), lambda i: (i, 0)),
        out_shape=jax.ShapeDtypeStruct((E_TOT, 128), jnp.float32),
    )(relt, msg, gtw, gtb, wem, wet)


def _max_body(a0, a1, mx_o):
    i = lax.broadcasted_iota(jnp.int32, (1, 128), 1)
    m0 = jnp.max(a0[...])
    m1 = jnp.max(a1[...])
    mx_o[...] = jnp.where(i == 0, m0, jnp.where(i == 1, m1, 0.0))


def _max_tc(a0r, a1r):
    full = lambda a: pl.BlockSpec(a.shape, lambda: (0,) * a.ndim)
    return pl.pallas_call(
        _max_body,
        in_specs=[full(a0r), full(a1r)],
        out_specs=pl.BlockSpec((1, 128), lambda: (0, 0)),
        out_shape=jax.ShapeDtypeStruct((1, 128), jnp.float32),
    )(a0r, a1r)


def _final_body(n0, n1, d0, d1, sk, out_o):
    num = n0[...] + n1[...]
    den0 = d0[:, 0:1] + d1[:, 0:1]
    den1 = d0[:, 1:2] + d1[:, 1:2]
    NB = num.shape[0]
    den = jnp.concatenate([jnp.broadcast_to(den0, (NB, 64)),
                           jnp.broadcast_to(den1, (NB, 64))], axis=1)
    out_o[...] = num / (den + 1e-16) + sk[...]


def _final_tc(num, den, sk, NB=512):
    grid = (NP // NB,)
    nblk = NP // NB
    return pl.pallas_call(
        _final_body,
        grid=grid,
        in_specs=[pl.BlockSpec((NB, 128), lambda i: (i, 0)),
                  pl.BlockSpec((NB, 128), lambda i, n=nblk: (i + n, 0)),
                  pl.BlockSpec((NB, 128), lambda i: (i, 0)),
                  pl.BlockSpec((NB, 128), lambda i, n=nblk: (i + n, 0)),
                  pl.BlockSpec((NB, 128), lambda i: (i, 0))],
        out_specs=pl.BlockSpec((NB, 128), lambda i: (i, 0)),
        out_shape=jax.ShapeDtypeStruct((NP, 128), jnp.float32),
    )(num, num, den, den, sk)


def kernel(mem, mem_msg, direction, msg, mt_w, mt_b, gru_w_ih, gru_w_hh,
           gru_b_ih, gru_b_hh, gt_w, gt_b, wq, bq, wk, bk, wv, bv, we, ws,
           bs, mem_ints, n_id, edge_index, t):
    Nn = n_id.shape[0]
    f32 = jnp.float32

    nid_pad = jnp.pad(n_id.astype(jnp.int32), (0, NP - Nn))
    small = jnp.pad(jnp.concatenate([direction, mem_ints], axis=1),
                    ((0, 0), (0, 123)))
    msgt = jnp.pad(mem_msg, ((0, 0), (0, 28)))
    luc = jnp.pad(mem_ints[:, 0], (0, LUT - mem_ints.shape[0]))
    didc = jnp.pad(mem_ints[:, 2], (0, LUT - mem_ints.shape[0]))
    src = edge_index[0].astype(jnp.int32)
    dst = edge_index[1].astype(jnp.int32)
    tf = t.astype(f32)

    small_g, srcm, dstm, raw, relt = _sc_gather(
        nid_pad, small, mem, msgt, n_id.astype(jnp.int32), luc, didc,
        src, tf)

    wih_t = gru_w_ih.T
    w3 = jnp.pad(wih_t[256:356], ((0, 28), (0, 0)))
    qn, kn, vn, sk = _node_dense(
        srcm, dstm, raw, small_g,
        wih_t[0:128], wih_t[128:256], w3, wih_t[356:456],
        gru_b_ih[None, :], gru_w_hh.T, gru_b_hh[None, :],
        mt_w.T, mt_b[None, :], wq.T, bq[None, :], wk.T, bk[None, :],
        wv.T, bv[None, :], ws.T, bs[None, :])

    e = _edge_dense(relt[:, None], msg, gt_w.T, gt_b[None, :],
                    we.T[0:100], we.T[100:200])

    a0, a1 = _sc_alpha(dst, src, qn, kn, e)

    mx = _max_tc(a0.reshape(2500, 128), a1.reshape(2500, 128))
    mx16 = mx[0, 0:16]

    num, den = _sc_scatter(dst, src, vn, e, a0, a1, mx16)

    out = _final_tc(num, den, sk)
    return out[:Nn]


# speedup vs baseline: 16.2509x; 2.3821x over previous
import functools

import jax
import jax.numpy as jnp
from jax import lax
from jax.experimental import pallas as pl
from jax.experimental.pallas import tpu as pltpu
from jax.experimental.pallas import tpu_sc as plsc

NC = 2
NS = 16
NW = NC * NS
NP = 10240
E_TOT = 320000
EW = E_TOT // NW
CH = 80
NCHUNK = EW // CH
LUT = 10008


def _iota16():
    return lax.iota(jnp.int32, 16)


def _full16(c):
    return jnp.full((16,), c, jnp.int32)


def _sc_gather_body(nid_ref, small_ref, mem_ref, msgt_ref, nidt_ref,
                    luc_ref, didc_ref, srce_ref, tf_ref,
                    small_o, srcm_o, dstm_o, raw_o, relt_o,
                    sidx_v, didx_v, small_v, rows_v, raw_v,
                    nid_v, luc_v, didc_v, eidx_v, tf_v, relt_v, sem):
    wid = lax.axis_index("s") * NC + lax.axis_index("c")
    pltpu.sync_copy(didc_ref, didc_v)

    for b in range(4):
        base = wid * 320 + b * 80
        pltpu.sync_copy(nid_ref.at[pl.ds(base, 80)], sidx_v)
        pltpu.async_copy(small_ref.at[sidx_v], small_v, sem).wait()
        pltpu.sync_copy(small_v, small_o.at[pl.ds(base, 80)])
        for g in range(5):
            nid16 = sidx_v[pl.ds(g * 16, 16)]
            did = plsc.load_gather(didc_v, [nid16])
            didx_v[pl.ds(g * 16, 16)] = did.astype(jnp.int32)
        pltpu.async_copy(mem_ref.at[sidx_v], rows_v, sem).wait()
        pltpu.sync_copy(rows_v, srcm_o.at[pl.ds(base, 80)])
        pltpu.async_copy(mem_ref.at[didx_v], rows_v, sem).wait()
        pltpu.sync_copy(rows_v, dstm_o.at[pl.ds(base, 80)])
        pltpu.async_copy(msgt_ref.at[sidx_v], raw_v, sem).wait()
        pltpu.sync_copy(raw_v, raw_o.at[pl.ds(base, 80)])

    pltpu.sync_copy(nidt_ref, nid_v)
    pltpu.sync_copy(luc_ref, luc_v)
    ECH = 2000

    def echunk(ch, _):
        ebase = wid * EW + ch * ECH
        pltpu.sync_copy(srce_ref.at[pl.ds(ebase, ECH)], eidx_v)
        pltpu.sync_copy(tf_ref.at[pl.ds(ebase, ECH)], tf_v)

        def grp(g, _):
            idx16 = eidx_v[pl.ds(g * 16, 16)]
            j16 = plsc.load_gather(nid_v, [idx16])
            lu16 = plsc.load_gather(luc_v, [j16])
            relt_v[pl.ds(g * 16, 16)] = lu16 - tf_v[pl.ds(g * 16, 16)]
            return 0

        lax.fori_loop(0, ECH // 16, grp, 0)
        pltpu.sync_copy(relt_v, relt_o.at[pl.ds(ebase, ECH)])
        return 0

    lax.fori_loop(0, EW // ECH, echunk, 0)


def _sc_gather(nid_pad, small, mem, msgt, nidt, luc, didc, src_e, tf):
    f32 = jnp.float32
    mesh = plsc.VectorSubcoreMesh(core_axis_name="c", subcore_axis_name="s")
    return pl.kernel(
        _sc_gather_body,
        out_type=[
            jax.ShapeDtypeStruct((NP, 128), f32),
            jax.ShapeDtypeStruct((NP, 128), f32),
            jax.ShapeDtypeStruct((NP, 128), f32),
            jax.ShapeDtypeStruct((NP, 128), f32),
            jax.ShapeDtypeStruct((E_TOT,), f32),
        ],
        mesh=mesh,
        compiler_params=pltpu.CompilerParams(needs_layout_passes=False),
        scratch_types=[
            pltpu.VMEM((80,), jnp.int32),
            pltpu.VMEM((80,), jnp.int32),
            pltpu.VMEM((80, 128), f32),
            pltpu.VMEM((80, 128), f32),
            pltpu.VMEM((80, 128), f32),
            pltpu.VMEM((10000,), jnp.int32),
            pltpu.VMEM((LUT,), f32),
            pltpu.VMEM((LUT,), f32),
            pltpu.VMEM((2000,), jnp.int32),
            pltpu.VMEM((2000,), f32),
            pltpu.VMEM((2000,), f32),
            pltpu.SemaphoreType.DMA,
        ],
    )(nid_pad, small, mem, msgt, nidt, luc, didc, src_e, tf)


def _sc_alpha_body(dst_ref, src_ref, qn_ref, kn_ref, e_ref,
                   a0_o, a1_o,
                   didx0, sidx0, q0, k0, e0,
                   didx1, sidx1, q1, k1, e1,
                   a0_v, a1_v, stage0, stage1,
                   semq0, semk0, seme0, semq1, semk1, seme1):
    wid = lax.axis_index("s") * NC + lax.axis_index("c")
    bufs = ((didx0, sidx0, q0, k0, e0, semq0, semk0, seme0),
            (didx1, sidx1, q1, k1, e1, semq1, semk1, seme1))

    def prefetch(ch, b):
        didx_v, sidx_v, q_v, k_v, e_v, sq, sk_, se = bufs[b]
        ebase = wid * EW + ch * CH
        pltpu.sync_copy(dst_ref.at[pl.ds(ebase, CH)], didx_v)
        pltpu.sync_copy(src_ref.at[pl.ds(ebase, CH)], sidx_v)
        pltpu.async_copy(qn_ref.at[didx_v], q_v, sq)
        pltpu.async_copy(kn_ref.at[sidx_v], k_v, sk_)
        pltpu.async_copy(e_ref.at[pl.ds(ebase, CH)], e_v, se)

    def wait(b):
        didx_v, sidx_v, q_v, k_v, e_v, sq, sk_, se = bufs[b]
        pltpu.make_async_copy(qn_ref.at[didx_v], q_v, sq).wait()
        pltpu.make_async_copy(kn_ref.at[sidx_v], k_v, sk_).wait()
        pltpu.make_async_copy(e_ref.at[pl.ds(0, CH)], e_v, se).wait()

    def compute(ch, b):
        _, _, q_v, k_v, e_v, _, _, _ = bufs[b]
        ebase = wid * EW + ch * CH
        for g in range(CH // 16):
            def edge2(i2, _):
                for u in range(2):
                    row = g * 16 + i2 * 2 + u
                    t = []
                    for j in range(8):
                        qj = q_v[row, pl.ds(j * 16, 16)]
                        kj = k_v[row, pl.ds(j * 16, 16)]
                        ej = e_v[row, pl.ds(j * 16, 16)]
                        t.append(qj * (kj + ej))
                    s0 = (t[0] + t[1]) + (t[2] + t[3])
                    s1 = (t[4] + t[5]) + (t[6] + t[7])
                    stage0[i2 * 2 + u, pl.ds(0, 16)] = s0
                    stage1[i2 * 2 + u, pl.ds(0, 16)] = s1
                return 0

            lax.fori_loop(0, 8, edge2, 0)
            acc0 = jnp.zeros((16,), jnp.float32)
            acc1 = jnp.zeros((16,), jnp.float32)
            for cc in range(16):
                acc0 = acc0 + plsc.load_gather(stage0,
                                               [_iota16(), _full16(cc)])
                acc1 = acc1 + plsc.load_gather(stage1,
                                               [_iota16(), _full16(cc)])
            a0_v[pl.ds(g * 16, 16)] = acc0 * 0.125
            a1_v[pl.ds(g * 16, 16)] = acc1 * 0.125
        pltpu.sync_copy(a0_v, a0_o.at[pl.ds(ebase, CH)])
        pltpu.sync_copy(a1_v, a1_o.at[pl.ds(ebase, CH)])

    prefetch(0, 0)

    def pair(p, _):
        ch = p * 2
        prefetch2 = ch + 1
        prefetch(prefetch2, 1)
        wait(0)
        compute(ch, 0)
        prefetch(ch + 2, 0)
        wait(1)
        compute(ch + 1, 1)
        return 0

    lax.fori_loop(0, (NCHUNK - 1) // 2, pair, 0)
    wait(0)
    compute(NCHUNK - 1, 0)


def _sc_alpha(dst, src, qn, kn, e):
    f32 = jnp.float32
    mesh = plsc.VectorSubcoreMesh(core_axis_name="c", subcore_axis_name="s")
    return pl.kernel(
        _sc_alpha_body,
        out_type=[jax.ShapeDtypeStruct((E_TOT,), f32),
                  jax.ShapeDtypeStruct((E_TOT,), f32)],
        mesh=mesh,
        compiler_params=pltpu.CompilerParams(needs_layout_passes=False),
        scratch_types=[
            pltpu.VMEM((CH,), jnp.int32),
            pltpu.VMEM((CH,), jnp.int32),
            pltpu.VMEM((CH, 128), f32),
            pltpu.VMEM((CH, 128), f32),
            pltpu.VMEM((CH, 128), f32),
            pltpu.VMEM((CH,), jnp.int32),
            pltpu.VMEM((CH,), jnp.int32),
            pltpu.VMEM((CH, 128), f32),
            pltpu.VMEM((CH, 128), f32),
            pltpu.VMEM((CH, 128), f32),
            pltpu.VMEM((CH,), f32),
            pltpu.VMEM((CH,), f32),
            pltpu.VMEM((16, 17), f32),
            pltpu.VMEM((16, 17), f32),
            pltpu.SemaphoreType.DMA,
            pltpu.SemaphoreType.DMA,
            pltpu.SemaphoreType.DMA,
            pltpu.SemaphoreType.DMA,
            pltpu.SemaphoreType.DMA,
            pltpu.SemaphoreType.DMA,
        ],
    )(dst, src, qn, kn, e)


def _sc_scatter_body(dst_ref, src_ref, vn_ref, e_ref, a0_ref, a1_ref,
                     mx_ref, num_o, den_o,
                     didx0, sidx0, v0, e0, a00, a10,
                     didx1, sidx1, v1, e1, a01, a11,
                     mx_v, wstage, acc_sh,
                     semv0, seme0, semsc0, semv1, seme1, semsc1):
    c = lax.axis_index("c")
    s = lax.axis_index("s")
    wid = s * NC + c
    z16 = jnp.zeros((16,), jnp.float32)
    bufs = ((didx0, sidx0, v0, e0, a00, a10, e0, semv0, seme0, semsc0),
            (didx1, sidx1, v1, e1, a01, a11, e1, semv1, seme1, semsc1))

    def zero_rows():
        def zrow(r, _):
            for cc in range(8):
                e0[r, pl.ds(cc * 16, 16)] = z16
                e1[r, pl.ds(cc * 16, 16)] = z16
            return 0

        lax.fori_loop(0, CH, zrow, 0)

    def zero_acc():
        for j in range(8):
            pltpu.sync_copy(e0, acc_sh.at[pl.ds(s * 640 + j * 80, 80)])

    zero_rows()
    zero_acc()
    pltpu.sync_copy(mx_ref, mx_v)
    plsc.subcore_barrier()

    mv = mx_v[pl.ds(0, 16)]
    mx0 = mv[0]
    mx1 = mv[1]

    def prefetch1(ch, b):
        didx_v, sidx_v, v_v, e_v, a0_v, a1_v, _, sv, se, _ = bufs[b]
        ebase = wid * EW + ch * CH
        pltpu.sync_copy(dst_ref.at[pl.ds(ebase, CH)], didx_v)
        pltpu.sync_copy(src_ref.at[pl.ds(ebase, CH)], sidx_v)
        pltpu.async_copy(vn_ref.at[sidx_v], v_v, sv)
        pltpu.async_copy(e_ref.at[pl.ds(ebase, CH)], e_v, se)
        pltpu.sync_copy(a0_ref.at[pl.ds(ebase, CH)], a0_v)
        pltpu.sync_copy(a1_ref.at[pl.ds(ebase, CH)], a1_v)

    def wait1(b):
        didx_v, sidx_v, v_v, e_v, _, _, _, sv, se, _ = bufs[b]
        pltpu.make_async_copy(vn_ref.at[sidx_v], v_v, sv).wait()
        pltpu.make_async_copy(e_ref.at[pl.ds(0, CH)], e_v, se).wait()

    def compute1(b):
        _, _, v_v, e_v, a0_v, a1_v, row_v, _, _, _ = bufs[b]
        for g in range(CH // 16):
            wstage[0, pl.ds(0, 16)] = jnp.exp(a0_v[pl.ds(g * 16, 16)] - mx0)
            wstage[1, pl.ds(0, 16)] = jnp.exp(a1_v[pl.ds(g * 16, 16)] - mx1)

            def edge2(i2, _):
                for u in range(2):
                    i = i2 * 2 + u
                    row = g * 16 + i
                    w0s = plsc.load_gather(wstage, [_full16(0), _full16(i)])
                    w1s = plsc.load_gather(wstage, [_full16(1), _full16(i)])
                    for j in range(8):
                        vj = v_v[row, pl.ds(j * 16, 16)]
                        ej = e_v[row, pl.ds(j * 16, 16)]
                        w = w0s if j < 4 else w1s
                        row_v[row, pl.ds(j * 16, 16)] = (vj + ej) * w
                return 0

            lax.fori_loop(0, 8, edge2, 0)

    def scatter(b):
        didx_v, _, _, _, _, _, row_v, _, _, ssc = bufs[b]
        pltpu.async_copy(row_v, acc_sh.at[didx_v], ssc, add=True)

    def wait_sc(b):
        didx_v, _, _, _, _, _, row_v, _, _, ssc = bufs[b]
        pltpu.make_async_copy(row_v, acc_sh.at[didx_v], ssc).wait()

    prefetch1(0, 0)

    def pair1(p, _):
        for u in range(2):
            ch = p * 2 + u
            wait1(u)
            compute1(u)
            scatter(u)
            if u == 0:
                pl.when(p >= 1)(lambda: wait_sc(1))
            else:
                wait_sc(0)
            prefetch1(ch + 1, 1 - u)
        return 0

    lax.fori_loop(0, (NCHUNK - 1) // 2, pair1, 0)
    wait1(0)
    compute1(0)
    scatter(0)
    wait_sc(1)
    wait_sc(0)
    plsc.subcore_barrier()
    for j in range(8):
        r0 = s * 640 + j * 80
        pltpu.sync_copy(acc_sh.at[pl.ds(r0, 80)], v0)
        pltpu.sync_copy(v0, num_o.at[pl.ds(c * NP + r0, 80)])
    plsc.subcore_barrier()

    zero_rows()
    zero_acc()
    plsc.subcore_barrier()

    def prefetch2(ch, b):
        didx_v, _, _, _, a0_v, a1_v, _, _, _, _ = bufs[b]
        ebase = wid * EW + ch * CH
        pltpu.sync_copy(dst_ref.at[pl.ds(ebase, CH)], didx_v)
        pltpu.sync_copy(a0_ref.at[pl.ds(ebase, CH)], a0_v)
        pltpu.sync_copy(a1_ref.at[pl.ds(ebase, CH)], a1_v)

    def compute2(b):
        _, _, _, _, a0_v, a1_v, row_v, _, _, _ = bufs[b]
        for g in range(CH // 16):
            rows16 = _iota16() + g * 16
            w0 = jnp.exp(a0_v[pl.ds(g * 16, 16)] - mx0)
            w1 = jnp.exp(a1_v[pl.ds(g * 16, 16)] - mx1)
            plsc.store_scatter(row_v, [rows16, _full16(0)], w0)
            plsc.store_scatter(row_v, [rows16, _full16(1)], w1)

    prefetch2(0, 0)

    def pair2(p, _):
        for u in range(2):
            ch = p * 2 + u
            compute2(u)
            scatter(u)
            if u == 0:
                pl.when(p >= 1)(lambda: wait_sc(1))
            else:
                wait_sc(0)
            prefetch2(ch + 1, 1 - u)
        return 0

    lax.fori_loop(0, (NCHUNK - 1) // 2, pair2, 0)
    compute2(0)
    scatter(0)
    wait_sc(1)
    wait_sc(0)
    plsc.subcore_barrier()
    for j in range(8):
        r0 = s * 640 + j * 80
        pltpu.sync_copy(acc_sh.at[pl.ds(r0, 80)], v0)
        pltpu.sync_copy(v0, den_o.at[pl.ds(c * NP + r0, 80)])


def _sc_scatter(dst, src, vn, e, a0, a1, mx16):
    f32 = jnp.float32
    i32 = jnp.int32
    mesh = plsc.VectorSubcoreMesh(core_axis_name="c", subcore_axis_name="s")
    buf = [pltpu.VMEM((CH,), i32), pltpu.VMEM((CH,), i32),
           pltpu.VMEM((CH, 128), f32), pltpu.VMEM((CH, 128), f32),
           pltpu.VMEM((CH,), f32), pltpu.VMEM((CH,), f32)]
    return pl.kernel(
        _sc_scatter_body,
        out_type=[jax.ShapeDtypeStruct((2 * NP, 128), f32),
                  jax.ShapeDtypeStruct((2 * NP, 128), f32)],
        mesh=mesh,
        compiler_params=pltpu.CompilerParams(needs_layout_passes=False),
        scratch_types=buf + buf + [
            pltpu.VMEM((16,), f32),
            pltpu.VMEM((2, 16), f32),
            pltpu.VMEM_SHARED((NP, 128), f32),
            pltpu.SemaphoreType.DMA,
            pltpu.SemaphoreType.DMA,
            pltpu.SemaphoreType.DMA,
            pltpu.SemaphoreType.DMA,
            pltpu.SemaphoreType.DMA,
            pltpu.SemaphoreType.DMA,
        ],
    )(dst, src, vn, e, a0, a1, mx16)


def _node_dense_body(srcm, dstm, raw, aux, w1, w2, w3, w4, bih, whh, bhh,
                     mtw, mtb, wq, bq, wk, bk, wv, bv, ws, bs,
                     qn_o, kn_o, vn_o, sk_o):
    d0 = aux[:, 0:1]
    d1 = aux[:, 1:2]
    rt = aux[:, 3:4]
    s = srcm[...]
    dm = dstm[...]
    m1 = s * d0 + dm * d1
    m2 = s * d1 + dm * d0
    te = jnp.cos(rt * mtw[...] + mtb[...])
    f32 = jnp.float32
    gi = (jnp.dot(m1, w1[...], preferred_element_type=f32)
          + jnp.dot(m2, w2[...], preferred_element_type=f32)
          + jnp.dot(raw[...], w3[...], preferred_element_type=f32)
          + jnp.dot(te, w4[...], preferred_element_type=f32) + bih[...])
    gh = jnp.dot(s, whh[...], preferred_element_type=f32) + bhh[...]
    M = 128
    r = jax.nn.sigmoid(gi[:, :M] + gh[:, :M])
    z = jax.nn.sigmoid(gi[:, M:2 * M] + gh[:, M:2 * M])
    n = jnp.tanh(gi[:, 2 * M:] + r * gh[:, 2 * M:])
    x = (1.0 - z) * n + z * s
    qn_o[...] = jnp.dot(x, wq[...], preferred_element_type=f32) + bq[...]
    kn_o[...] = jnp.dot(x, wk[...], preferred_element_type=f32) + bk[...]
    vn_o[...] = jnp.dot(x, wv[...], preferred_element_type=f32) + bv[...]
    sk_o[...] = jnp.dot(x, ws[...], preferred_element_type=f32) + bs[...]


def _node_dense(srcm, dstm, raw, aux, w1, w2, w3, w4, bih, whh, bhh,
                mtw, mtb, wq, bq, wk, bk, wv, bv, ws, bs, NB=512):
    grid = (NP // NB,)
    row_spec = lambda c: pl.BlockSpec((NB, c), lambda i: (i, 0))
    full = lambda a: pl.BlockSpec(a.shape, lambda i: (0,) * a.ndim)
    out_shape = [jax.ShapeDtypeStruct((NP, 128), jnp.float32)] * 4
    return pl.pallas_call(
        _node_dense_body,
        grid=grid,
        in_specs=[row_spec(128), row_spec(128), row_spec(128), row_spec(128)]
        + [full(a) for a in (w1, w2, w3, w4, bih, whh, bhh, mtw, mtb,
                             wq, bq, wk, bk, wv, bv, ws, bs)],
        out_specs=[row_spec(128)] * 4,
        out_shape=out_shape,
    )(srcm, dstm, raw, aux, w1, w2, w3, w4, bih, whh, bhh, mtw, mtb,
      wq, bq, wk, bk, wv, bv, ws, bs)


def _edge_dense_body(relt, msg, gtw, gtb, wem, wet, e_o):
    te = jnp.cos(relt[...] * gtw[...] + gtb[...])
    f32 = jnp.float32
    e_o[...] = (jnp.dot(msg[...], wem[...], preferred_element_type=f32)
                + jnp.dot(te, wet[...], preferred_element_type=f32))


def _edge_dense(relt, msg, gtw, gtb, wem, wet, EB=2560):
    grid = (E_TOT // EB,)
    full = lambda a: pl.BlockSpec(a.shape, lambda i: (0,) * a.ndim)
    return pl.pallas_call(
        _edge_dense_body,
        grid=grid,
        in_specs=[pl.BlockSpec((EB, 1), lambda i: (i, 0)),
                  pl.BlockSpec((EB, 100), lambda i: (i, 0)),
                  full(gtw), full(gtb), full(wem), full(wet)],
        out_specs=pl.BlockSpec((EB, 128), lambda i: (i, 0)),
        out_shape=jax.ShapeDtypeStruct((E_TOT, 128), jnp.float32),
    )(relt, msg, gtw, gtb, wem, wet)


def _max_body(a0, a1, mx_o):
    i = lax.broadcasted_iota(jnp.int32, (1, 128), 1)
    m0 = jnp.max(a0[...])
    m1 = jnp.max(a1[...])
    mx_o[...] = jnp.where(i == 0, m0, jnp.where(i == 1, m1, 0.0))


def _max_tc(a0r, a1r):
    full = lambda a: pl.BlockSpec(a.shape, lambda: (0,) * a.ndim)
    return pl.pallas_call(
        _max_body,
        in_specs=[full(a0r), full(a1r)],
        out_specs=pl.BlockSpec((1, 128), lambda: (0, 0)),
        out_shape=jax.ShapeDtypeStruct((1, 128), jnp.float32),
    )(a0r, a1r)


def _final_body(n0, n1, d0, d1, sk, out_o):
    num = n0[...] + n1[...]
    den0 = d0[:, 0:1] + d1[:, 0:1]
    den1 = d0[:, 1:2] + d1[:, 1:2]
    NB = num.shape[0]
    den = jnp.concatenate([jnp.broadcast_to(den0, (NB, 64)),
                           jnp.broadcast_to(den1, (NB, 64))], axis=1)
    out_o[...] = num / (den + 1e-16) + sk[...]


def _final_tc(num, den, sk, NB=512):
    grid = (NP // NB,)
    nblk = NP // NB
    return pl.pallas_call(
        _final_body,
        grid=grid,
        in_specs=[pl.BlockSpec((NB, 128), lambda i: (i, 0)),
                  pl.BlockSpec((NB, 128), lambda i, n=nblk: (i + n, 0)),
                  pl.BlockSpec((NB, 128), lambda i: (i, 0)),
                  pl.BlockSpec((NB, 128), lambda i, n=nblk: (i + n, 0)),
                  pl.BlockSpec((NB, 128), lambda i: (i, 0))],
        out_specs=pl.BlockSpec((NB, 128), lambda i: (i, 0)),
        out_shape=jax.ShapeDtypeStruct((NP, 128), jnp.float32),
    )(num, num, den, den, sk)


def kernel(mem, mem_msg, direction, msg, mt_w, mt_b, gru_w_ih, gru_w_hh,
           gru_b_ih, gru_b_hh, gt_w, gt_b, wq, bq, wk, bk, wv, bv, we, ws,
           bs, mem_ints, n_id, edge_index, t):
    Nn = n_id.shape[0]
    f32 = jnp.float32

    nid_pad = jnp.pad(n_id.astype(jnp.int32), (0, NP - Nn))
    small = jnp.pad(jnp.concatenate([direction, mem_ints], axis=1),
                    ((0, 0), (0, 123)))
    msgt = jnp.pad(mem_msg, ((0, 0), (0, 28)))
    luc = jnp.pad(mem_ints[:, 0], (0, LUT - mem_ints.shape[0]))
    didc = jnp.pad(mem_ints[:, 2], (0, LUT - mem_ints.shape[0]))
    src = edge_index[0].astype(jnp.int32)
    dst = edge_index[1].astype(jnp.int32)
    tf = t.astype(f32)

    small_g, srcm, dstm, raw, relt = _sc_gather(
        nid_pad, small, mem, msgt, n_id.astype(jnp.int32), luc, didc,
        src, tf)

    wih_t = gru_w_ih.T
    w3 = jnp.pad(wih_t[256:356], ((0, 28), (0, 0)))
    qn, kn, vn, sk = _node_dense(
        srcm, dstm, raw, small_g,
        wih_t[0:128], wih_t[128:256], w3, wih_t[356:456],
        gru_b_ih[None, :], gru_w_hh.T, gru_b_hh[None, :],
        mt_w.T, mt_b[None, :], wq.T, bq[None, :], wk.T, bk[None, :],
        wv.T, bv[None, :], ws.T, bs[None, :])

    e = _edge_dense(relt[:, None], msg, gt_w.T, gt_b[None, :],
                    we.T[0:100], we.T[100:200])

    a0, a1 = _sc_alpha(dst, src, qn, kn, e)

    mx = _max_tc(a0.reshape(2500, 128), a1.reshape(2500, 128))
    mx16 = mx[0, 0:16]

    num, den = _sc_scatter(dst, src, vn, e, a0, a1, mx16)

    out = _final_tc(num, den, sk)
    return out[:Nn]


# speedup vs baseline: 17.3599x; 1.0682x over previous
import functools

import jax
import jax.numpy as jnp
from jax import lax
from jax.experimental import pallas as pl
from jax.experimental.pallas import tpu as pltpu
from jax.experimental.pallas import tpu_sc as plsc

NC = 2
NS = 16
NW = NC * NS
NP = 10240
E_TOT = 320000
EW = E_TOT // NW
CH = 80
NCHUNK = EW // CH
LUT = 10008


def _iota16():
    return lax.iota(jnp.int32, 16)


def _full16(c):
    return jnp.full((16,), c, jnp.int32)


def _sc_gather_body(nid_ref, small_ref, mem_ref, msgt_ref, nidt_ref,
                    luc_ref, didc_ref, srce_ref, tf_ref,
                    small_o, srcm_o, dstm_o, raw_o, relt_o,
                    sidx_v, didx_v, small_v, rows_v, raw_v,
                    nid_v, luc_v, didc_v, eidx_v, tf_v, relt_v, sem):
    wid = lax.axis_index("s") * NC + lax.axis_index("c")
    pltpu.sync_copy(didc_ref, didc_v)

    for b in range(4):
        base = wid * 320 + b * 80
        pltpu.sync_copy(nid_ref.at[pl.ds(base, 80)], sidx_v)
        pltpu.async_copy(small_ref.at[sidx_v], small_v, sem).wait()
        pltpu.sync_copy(small_v, small_o.at[pl.ds(base, 80)])
        for g in range(5):
            nid16 = sidx_v[pl.ds(g * 16, 16)]
            did = plsc.load_gather(didc_v, [nid16])
            didx_v[pl.ds(g * 16, 16)] = did.astype(jnp.int32)
        pltpu.async_copy(mem_ref.at[sidx_v], rows_v, sem).wait()
        pltpu.sync_copy(rows_v, srcm_o.at[pl.ds(base, 80)])
        pltpu.async_copy(mem_ref.at[didx_v], rows_v, sem).wait()
        pltpu.sync_copy(rows_v, dstm_o.at[pl.ds(base, 80)])
        pltpu.async_copy(msgt_ref.at[sidx_v], raw_v, sem).wait()
        pltpu.sync_copy(raw_v, raw_o.at[pl.ds(base, 80)])

    pltpu.sync_copy(nidt_ref, nid_v)
    pltpu.sync_copy(luc_ref, luc_v)
    ECH = 2000

    def echunk(ch, _):
        ebase = wid * EW + ch * ECH
        pltpu.sync_copy(srce_ref.at[pl.ds(ebase, ECH)], eidx_v)
        pltpu.sync_copy(tf_ref.at[pl.ds(ebase, ECH)], tf_v)

        def grp(g, _):
            idx16 = eidx_v[pl.ds(g * 16, 16)]
            j16 = plsc.load_gather(nid_v, [idx16])
            lu16 = plsc.load_gather(luc_v, [j16])
            relt_v[pl.ds(g * 16, 16)] = lu16 - tf_v[pl.ds(g * 16, 16)]
            return 0

        lax.fori_loop(0, ECH // 16, grp, 0)
        pltpu.sync_copy(relt_v, relt_o.at[pl.ds(ebase, ECH)])
        return 0

    lax.fori_loop(0, EW // ECH, echunk, 0)


def _sc_gather(nid_pad, small, mem, msgt, nidt, luc, didc, src_e, tf):
    f32 = jnp.float32
    mesh = plsc.VectorSubcoreMesh(core_axis_name="c", subcore_axis_name="s")
    return pl.kernel(
        _sc_gather_body,
        out_type=[
            jax.ShapeDtypeStruct((NP, 128), f32),
            jax.ShapeDtypeStruct((NP, 128), f32),
            jax.ShapeDtypeStruct((NP, 128), f32),
            jax.ShapeDtypeStruct((NP, 128), f32),
            jax.ShapeDtypeStruct((E_TOT,), f32),
        ],
        mesh=mesh,
        compiler_params=pltpu.CompilerParams(needs_layout_passes=False),
        scratch_types=[
            pltpu.VMEM((80,), jnp.int32),
            pltpu.VMEM((80,), jnp.int32),
            pltpu.VMEM((80, 128), f32),
            pltpu.VMEM((80, 128), f32),
            pltpu.VMEM((80, 128), f32),
            pltpu.VMEM((10000,), jnp.int32),
            pltpu.VMEM((LUT,), f32),
            pltpu.VMEM((LUT,), f32),
            pltpu.VMEM((2000,), jnp.int32),
            pltpu.VMEM((2000,), f32),
            pltpu.VMEM((2000,), f32),
            pltpu.SemaphoreType.DMA,
        ],
    )(nid_pad, small, mem, msgt, nidt, luc, didc, src_e, tf)


def _sc_alpha_body(dst_ref, src_ref, qn_ref, kn_ref, e_ref,
                   a0_o, a1_o,
                   didx0, sidx0, q0, k0, e0,
                   didx1, sidx1, q1, k1, e1,
                   a0_v, a1_v, stage0, stage1,
                   semq0, semk0, seme0, semq1, semk1, seme1):
    wid = lax.axis_index("s") * NC + lax.axis_index("c")
    bufs = ((didx0, sidx0, q0, k0, e0, semq0, semk0, seme0),
            (didx1, sidx1, q1, k1, e1, semq1, semk1, seme1))

    def prefetch(ch, b):
        didx_v, sidx_v, q_v, k_v, e_v, sq, sk_, se = bufs[b]
        ebase = wid * EW + ch * CH
        pltpu.sync_copy(dst_ref.at[pl.ds(ebase, CH)], didx_v)
        pltpu.sync_copy(src_ref.at[pl.ds(ebase, CH)], sidx_v)
        pltpu.async_copy(qn_ref.at[didx_v], q_v, sq)
        pltpu.async_copy(kn_ref.at[sidx_v], k_v, sk_)
        pltpu.async_copy(e_ref.at[pl.ds(ebase, CH)], e_v, se)

    def wait(b):
        didx_v, sidx_v, q_v, k_v, e_v, sq, sk_, se = bufs[b]
        pltpu.make_async_copy(qn_ref.at[didx_v], q_v, sq).wait()
        pltpu.make_async_copy(kn_ref.at[sidx_v], k_v, sk_).wait()
        pltpu.make_async_copy(e_ref.at[pl.ds(0, CH)], e_v, se).wait()

    def compute(ch, b):
        _, _, q_v, k_v, e_v, _, _, _ = bufs[b]
        ebase = wid * EW + ch * CH
        for g in range(CH // 16):
            def edge2(i2, _):
                for u in range(2):
                    row = g * 16 + i2 * 2 + u
                    t = []
                    for j in range(8):
                        qj = q_v[row, pl.ds(j * 16, 16)]
                        kj = k_v[row, pl.ds(j * 16, 16)]
                        ej = e_v[row, pl.ds(j * 16, 16)]
                        t.append(qj * (kj + ej))
                    s0 = (t[0] + t[1]) + (t[2] + t[3])
                    s1 = (t[4] + t[5]) + (t[6] + t[7])
                    stage0[i2 * 2 + u, pl.ds(0, 16)] = s0
                    stage1[i2 * 2 + u, pl.ds(0, 16)] = s1
                return 0

            lax.fori_loop(0, 8, edge2, 0)
            acc0 = jnp.zeros((16,), jnp.float32)
            acc1 = jnp.zeros((16,), jnp.float32)
            for cc in range(16):
                acc0 = acc0 + plsc.load_gather(stage0,
                                               [_iota16(), _full16(cc)])
                acc1 = acc1 + plsc.load_gather(stage1,
                                               [_iota16(), _full16(cc)])
            a0_v[pl.ds(g * 16, 16)] = acc0 * 0.125
            a1_v[pl.ds(g * 16, 16)] = acc1 * 0.125
        pltpu.sync_copy(a0_v, a0_o.at[pl.ds(ebase, CH)])
        pltpu.sync_copy(a1_v, a1_o.at[pl.ds(ebase, CH)])

    prefetch(0, 0)

    def pair(p, _):
        ch = p * 2
        prefetch2 = ch + 1
        prefetch(prefetch2, 1)
        wait(0)
        compute(ch, 0)
        prefetch(ch + 2, 0)
        wait(1)
        compute(ch + 1, 1)
        return 0

    lax.fori_loop(0, (NCHUNK - 1) // 2, pair, 0)
    wait(0)
    compute(NCHUNK - 1, 0)


def _sc_alpha(dst, src, qn, kn, e):
    f32 = jnp.float32
    mesh = plsc.VectorSubcoreMesh(core_axis_name="c", subcore_axis_name="s")
    return pl.kernel(
        _sc_alpha_body,
        out_type=[jax.ShapeDtypeStruct((E_TOT,), f32),
                  jax.ShapeDtypeStruct((E_TOT,), f32)],
        mesh=mesh,
        compiler_params=pltpu.CompilerParams(needs_layout_passes=False),
        scratch_types=[
            pltpu.VMEM((CH,), jnp.int32),
            pltpu.VMEM((CH,), jnp.int32),
            pltpu.VMEM((CH, 128), f32),
            pltpu.VMEM((CH, 128), f32),
            pltpu.VMEM((CH, 128), f32),
            pltpu.VMEM((CH,), jnp.int32),
            pltpu.VMEM((CH,), jnp.int32),
            pltpu.VMEM((CH, 128), f32),
            pltpu.VMEM((CH, 128), f32),
            pltpu.VMEM((CH, 128), f32),
            pltpu.VMEM((CH,), f32),
            pltpu.VMEM((CH,), f32),
            pltpu.VMEM((16, 17), f32),
            pltpu.VMEM((16, 17), f32),
            pltpu.SemaphoreType.DMA,
            pltpu.SemaphoreType.DMA,
            pltpu.SemaphoreType.DMA,
            pltpu.SemaphoreType.DMA,
            pltpu.SemaphoreType.DMA,
            pltpu.SemaphoreType.DMA,
        ],
    )(dst, src, qn, kn, e)


def _sc_scatter_body(dst_ref, src_ref, vn_ref, e_ref, a0_ref, a1_ref,
                     mx_ref, num_o,
                     didx0, sidx0, v0, e0, a00, a10,
                     didx1, sidx1, v1, e1, a01, a11,
                     mx_v, wstage, acc_sh,
                     semv0, seme0, semsc0, semv1, seme1, semsc1):
    c = lax.axis_index("c")
    s = lax.axis_index("s")
    wid = s * NC + c
    z16 = jnp.zeros((16,), jnp.float32)
    bufs = ((didx0, sidx0, v0, e0, a00, a10, e0, semv0, seme0, semsc0),
            (didx1, sidx1, v1, e1, a01, a11, e1, semv1, seme1, semsc1))

    def zero_rows():
        def zrow(r, _):
            for cc in range(8):
                e0[r, pl.ds(cc * 16, 16)] = z16
                e1[r, pl.ds(cc * 16, 16)] = z16
            return 0

        lax.fori_loop(0, CH, zrow, 0)

    def zero_acc():
        for j in range(8):
            pltpu.sync_copy(e0, acc_sh.at[pl.ds(s * 640 + j * 80, 80)])

    zero_rows()
    zero_acc()
    pltpu.sync_copy(mx_ref, mx_v)
    plsc.subcore_barrier()

    mv = mx_v[pl.ds(0, 16)]
    mx0 = mv[0]
    mx1 = mv[1]

    def prefetch1(ch, b):
        didx_v, sidx_v, v_v, e_v, a0_v, a1_v, _, sv, se, _ = bufs[b]
        ebase = wid * EW + ch * CH
        pltpu.sync_copy(dst_ref.at[pl.ds(ebase, CH)], didx_v)
        pltpu.sync_copy(src_ref.at[pl.ds(ebase, CH)], sidx_v)
        pltpu.async_copy(vn_ref.at[sidx_v], v_v, sv)
        pltpu.async_copy(e_ref.at[pl.ds(ebase, CH)], e_v, se)
        pltpu.sync_copy(a0_ref.at[pl.ds(ebase, CH)], a0_v)
        pltpu.sync_copy(a1_ref.at[pl.ds(ebase, CH)], a1_v)

    def wait1(b):
        didx_v, sidx_v, v_v, e_v, _, _, _, sv, se, _ = bufs[b]
        pltpu.make_async_copy(vn_ref.at[sidx_v], v_v, sv).wait()
        pltpu.make_async_copy(e_ref.at[pl.ds(0, CH)], e_v, se).wait()

    def compute1(b):
        _, _, v_v, e_v, a0_v, a1_v, row_v, _, _, _ = bufs[b]
        for g in range(CH // 16):
            wstage[0, pl.ds(0, 16)] = jnp.exp(a0_v[pl.ds(g * 16, 16)] - mx0)
            wstage[1, pl.ds(0, 16)] = jnp.exp(a1_v[pl.ds(g * 16, 16)] - mx1)

            def edge2(i2, _):
                for u in range(2):
                    i = i2 * 2 + u
                    row = g * 16 + i
                    w0s = plsc.load_gather(wstage, [_full16(0), _full16(i)])
                    w1s = plsc.load_gather(wstage, [_full16(1), _full16(i)])
                    for j in range(8):
                        vj = v_v[row, pl.ds(j * 16, 16)]
                        ej = e_v[row, pl.ds(j * 16, 16)]
                        w = w0s if j < 4 else w1s
                        row_v[row, pl.ds(j * 16, 16)] = (vj + ej) * w
                return 0

            lax.fori_loop(0, 8, edge2, 0)

    def scatter(b):
        didx_v, _, _, _, _, _, row_v, _, _, ssc = bufs[b]
        pltpu.async_copy(row_v, acc_sh.at[didx_v], ssc, add=True)

    def wait_sc(b):
        didx_v, _, _, _, _, _, row_v, _, _, ssc = bufs[b]
        pltpu.make_async_copy(row_v, acc_sh.at[didx_v], ssc).wait()

    prefetch1(0, 0)

    def pair1(p, _):
        for u in range(2):
            ch = p * 2 + u
            wait1(u)
            compute1(u)
            scatter(u)
            if u == 0:
                pl.when(p >= 1)(lambda: wait_sc(1))
            else:
                wait_sc(0)
            prefetch1(ch + 1, 1 - u)
        return 0

    lax.fori_loop(0, (NCHUNK - 1) // 2, pair1, 0)
    wait1(0)
    compute1(0)
    scatter(0)
    wait_sc(1)
    wait_sc(0)
    plsc.subcore_barrier()
    for j in range(8):
        r0 = s * 640 + j * 80
        pltpu.sync_copy(acc_sh.at[pl.ds(r0, 80)], v0)
        pltpu.sync_copy(v0, num_o.at[pl.ds(c * NP + r0, 80)])
    plsc.subcore_barrier()


def _sc_scatter(dst, src, vn, e, a0, a1, mx16):
    f32 = jnp.float32
    i32 = jnp.int32
    mesh = plsc.VectorSubcoreMesh(core_axis_name="c", subcore_axis_name="s")
    buf = [pltpu.VMEM((CH,), i32), pltpu.VMEM((CH,), i32),
           pltpu.VMEM((CH, 128), f32), pltpu.VMEM((CH, 128), f32),
           pltpu.VMEM((CH,), f32), pltpu.VMEM((CH,), f32)]
    return pl.kernel(
        _sc_scatter_body,
        out_type=jax.ShapeDtypeStruct((2 * NP, 128), f32),
        mesh=mesh,
        compiler_params=pltpu.CompilerParams(needs_layout_passes=False),
        scratch_types=buf + buf + [
            pltpu.VMEM((16,), f32),
            pltpu.VMEM((2, 16), f32),
            pltpu.VMEM_SHARED((NP, 128), f32),
            pltpu.SemaphoreType.DMA,
            pltpu.SemaphoreType.DMA,
            pltpu.SemaphoreType.DMA,
            pltpu.SemaphoreType.DMA,
            pltpu.SemaphoreType.DMA,
            pltpu.SemaphoreType.DMA,
        ],
    )(dst, src, vn, e, a0, a1, mx16)


def _sc_den_body(dst_ref, a0_ref, a1_ref, mx_ref, den0_o, den1_o,
                 didx_v, a0_v, a1_v, den0_v, den1_v, mx_v):
    wid = lax.axis_index("s") * NC + lax.axis_index("c")
    z16 = jnp.zeros((16,), jnp.float32)

    def zrow(r, _):
        den0_v[pl.ds(r * 16, 16)] = z16
        den1_v[pl.ds(r * 16, 16)] = z16
        return 0

    lax.fori_loop(0, NP // 16, zrow, 0)
    pltpu.sync_copy(mx_ref, mx_v)
    mv = mx_v[pl.ds(0, 16)]
    mx0 = mv[0]
    mx1 = mv[1]
    ECH = 2000

    def chunk(ch, _):
        ebase = wid * EW + ch * ECH
        pltpu.sync_copy(dst_ref.at[pl.ds(ebase, ECH)], didx_v)
        pltpu.sync_copy(a0_ref.at[pl.ds(ebase, ECH)], a0_v)
        pltpu.sync_copy(a1_ref.at[pl.ds(ebase, ECH)], a1_v)

        def grp(g, _):
            idx16 = didx_v[pl.ds(g * 16, 16)]
            w0 = jnp.exp(a0_v[pl.ds(g * 16, 16)] - mx0)
            w1 = jnp.exp(a1_v[pl.ds(g * 16, 16)] - mx1)
            plsc.addupdate_scatter(den0_v, [idx16], w0)
            plsc.addupdate_scatter(den1_v, [idx16], w1)
            return 0

        lax.fori_loop(0, ECH // 16, grp, 0)
        return 0

    lax.fori_loop(0, EW // ECH, chunk, 0)
    pltpu.sync_copy(den0_v, den0_o.at[wid])
    pltpu.sync_copy(den1_v, den1_o.at[wid])


def _sc_den(dst, a0, a1, mx16):
    f32 = jnp.float32
    mesh = plsc.VectorSubcoreMesh(core_axis_name="c", subcore_axis_name="s")
    return pl.kernel(
        _sc_den_body,
        out_type=[jax.ShapeDtypeStruct((NW, NP), f32),
                  jax.ShapeDtypeStruct((NW, NP), f32)],
        mesh=mesh,
        compiler_params=pltpu.CompilerParams(needs_layout_passes=False),
        scratch_types=[
            pltpu.VMEM((2000,), jnp.int32),
            pltpu.VMEM((2000,), f32),
            pltpu.VMEM((2000,), f32),
            pltpu.VMEM((NP,), f32),
            pltpu.VMEM((NP,), f32),
            pltpu.VMEM((16,), f32),
        ],
    )(dst, a0, a1, mx16)


def _node_dense_body(srcm, dstm, raw, aux, w1, w2, w3, w4, bih, whh, bhh,
                     mtw, mtb, wq, bq, wk, bk, wv, bv, ws, bs,
                     qn_o, kn_o, vn_o, sk_o):
    d0 = aux[:, 0:1]
    d1 = aux[:, 1:2]
    rt = aux[:, 3:4]
    s = srcm[...]
    dm = dstm[...]
    m1 = s * d0 + dm * d1
    m2 = s * d1 + dm * d0
    te = jnp.cos(rt * mtw[...] + mtb[...])
    f32 = jnp.float32
    gi = (jnp.dot(m1, w1[...], preferred_element_type=f32)
          + jnp.dot(m2, w2[...], preferred_element_type=f32)
          + jnp.dot(raw[...], w3[...], preferred_element_type=f32)
          + jnp.dot(te, w4[...], preferred_element_type=f32) + bih[...])
    gh = jnp.dot(s, whh[...], preferred_element_type=f32) + bhh[...]
    M = 128
    r = jax.nn.sigmoid(gi[:, :M] + gh[:, :M])
    z = jax.nn.sigmoid(gi[:, M:2 * M] + gh[:, M:2 * M])
    n = jnp.tanh(gi[:, 2 * M:] + r * gh[:, 2 * M:])
    x = (1.0 - z) * n + z * s
    qn_o[...] = jnp.dot(x, wq[...], preferred_element_type=f32) + bq[...]
    kn_o[...] = jnp.dot(x, wk[...], preferred_element_type=f32) + bk[...]
    vn_o[...] = jnp.dot(x, wv[...], preferred_element_type=f32) + bv[...]
    sk_o[...] = jnp.dot(x, ws[...], preferred_element_type=f32) + bs[...]


def _node_dense(srcm, dstm, raw, aux, w1, w2, w3, w4, bih, whh, bhh,
                mtw, mtb, wq, bq, wk, bk, wv, bv, ws, bs, NB=512):
    grid = (NP // NB,)
    row_spec = lambda c: pl.BlockSpec((NB, c), lambda i: (i, 0))
    full = lambda a: pl.BlockSpec(a.shape, lambda i: (0,) * a.ndim)
    out_shape = [jax.ShapeDtypeStruct((NP, 128), jnp.float32)] * 4
    return pl.pallas_call(
        _node_dense_body,
        grid=grid,
        in_specs=[row_spec(128), row_spec(128), row_spec(128), row_spec(128)]
        + [full(a) for a in (w1, w2, w3, w4, bih, whh, bhh, mtw, mtb,
                             wq, bq, wk, bk, wv, bv, ws, bs)],
        out_specs=[row_spec(128)] * 4,
        out_shape=out_shape,
    )(srcm, dstm, raw, aux, w1, w2, w3, w4, bih, whh, bhh, mtw, mtb,
      wq, bq, wk, bk, wv, bv, ws, bs)


def _edge_dense_body(relt, msg, gtw, gtb, wem, wet, e_o):
    te = jnp.cos(relt[...] * gtw[...] + gtb[...])
    f32 = jnp.float32
    e_o[...] = (jnp.dot(msg[...], wem[...], preferred_element_type=f32)
                + jnp.dot(te, wet[...], preferred_element_type=f32))


def _edge_dense(relt, msg, gtw, gtb, wem, wet, EB=2560):
    grid = (E_TOT // EB,)
    full = lambda a: pl.BlockSpec(a.shape, lambda i: (0,) * a.ndim)
    return pl.pallas_call(
        _edge_dense_body,
        grid=grid,
        in_specs=[pl.BlockSpec((EB, 1), lambda i: (i, 0)),
                  pl.BlockSpec((EB, 100), lambda i: (i, 0)),
                  full(gtw), full(gtb), full(wem), full(wet)],
        out_specs=pl.BlockSpec((EB, 128), lambda i: (i, 0)),
        out_shape=jax.ShapeDtypeStruct((E_TOT, 128), jnp.float32),
    )(relt, msg, gtw, gtb, wem, wet)


def _max_body(a0, a1, mx_o):
    i = lax.broadcasted_iota(jnp.int32, (1, 128), 1)
    m0 = jnp.max(a0[...])
    m1 = jnp.max(a1[...])
    mx_o[...] = jnp.where(i == 0, m0, jnp.where(i == 1, m1, 0.0))


def _max_tc(a0r, a1r):
    full = lambda a: pl.BlockSpec(a.shape, lambda: (0,) * a.ndim)
    return pl.pallas_call(
        _max_body,
        in_specs=[full(a0r), full(a1r)],
        out_specs=pl.BlockSpec((1, 128), lambda: (0, 0)),
        out_shape=jax.ShapeDtypeStruct((1, 128), jnp.float32),
    )(a0r, a1r)


def _final_body(n0, n1, d0, d1, sk, out_o):
    num = n0[...] + n1[...]
    den0 = jnp.sum(d0[...], axis=1, keepdims=True)
    den1 = jnp.sum(d1[...], axis=1, keepdims=True)
    NB = num.shape[0]
    den = jnp.concatenate([jnp.broadcast_to(den0, (NB, 64)),
                           jnp.broadcast_to(den1, (NB, 64))], axis=1)
    out_o[...] = num / (den + 1e-16) + sk[...]


def _final_tc(num, den0t, den1t, sk, NB=512):
    grid = (NP // NB,)
    nblk = NP // NB
    return pl.pallas_call(
        _final_body,
        grid=grid,
        in_specs=[pl.BlockSpec((NB, 128), lambda i: (i, 0)),
                  pl.BlockSpec((NB, 128), lambda i, n=nblk: (i + n, 0)),
                  pl.BlockSpec((NB, NW), lambda i: (i, 0)),
                  pl.BlockSpec((NB, NW), lambda i: (i, 0)),
                  pl.BlockSpec((NB, 128), lambda i: (i, 0))],
        out_specs=pl.BlockSpec((NB, 128), lambda i: (i, 0)),
        out_shape=jax.ShapeDtypeStruct((NP, 128), jnp.float32),
    )(num, num, den0t, den1t, sk)


def kernel(mem, mem_msg, direction, msg, mt_w, mt_b, gru_w_ih, gru_w_hh,
           gru_b_ih, gru_b_hh, gt_w, gt_b, wq, bq, wk, bk, wv, bv, we, ws,
           bs, mem_ints, n_id, edge_index, t):
    Nn = n_id.shape[0]
    f32 = jnp.float32

    nid_pad = jnp.pad(n_id.astype(jnp.int32), (0, NP - Nn))
    small = jnp.pad(jnp.concatenate([direction, mem_ints], axis=1),
                    ((0, 0), (0, 123)))
    msgt = jnp.pad(mem_msg, ((0, 0), (0, 28)))
    luc = jnp.pad(mem_ints[:, 0], (0, LUT - mem_ints.shape[0]))
    didc = jnp.pad(mem_ints[:, 2], (0, LUT - mem_ints.shape[0]))
    src = edge_index[0].astype(jnp.int32)
    dst = edge_index[1].astype(jnp.int32)
    tf = t.astype(f32)

    small_g, srcm, dstm, raw, relt = _sc_gather(
        nid_pad, small, mem, msgt, n_id.astype(jnp.int32), luc, didc,
        src, tf)

    wih_t = gru_w_ih.T
    w3 = jnp.pad(wih_t[256:356], ((0, 28), (0, 0)))
    qn, kn, vn, sk = _node_dense(
        srcm, dstm, raw, small_g,
        wih_t[0:128], wih_t[128:256], w3, wih_t[356:456],
        gru_b_ih[None, :], gru_w_hh.T, gru_b_hh[None, :],
        mt_w.T, mt_b[None, :], wq.T, bq[None, :], wk.T, bk[None, :],
        wv.T, bv[None, :], ws.T, bs[None, :])

    e = _edge_dense(relt[:, None], msg, gt_w.T, gt_b[None, :],
                    we.T[0:100], we.T[100:200])

    a0, a1 = _sc_alpha(dst, src, qn, kn, e)

    mx = _max_tc(a0.reshape(2500, 128), a1.reshape(2500, 128))
    mx16 = mx[0, 0:16]

    num = _sc_scatter(dst, src, vn, e, a0, a1, mx16)
    den0p, den1p = _sc_den(dst, a0, a1, mx16)

    out = _final_tc(num, den0p.T, den1p.T, sk)
    return out[:Nn]
